# Initial kernel scaffold; baseline (speedup 1.0000x reference)
import jax
import jax.numpy as jnp
from jax.experimental import pallas as pl


def kernel(x, hyperedge_index, hyperedge_weight, batch, theta1, hb1, gamma1, beta1, wlin1, blin1, wrel1, brel1, wroot1, theta2, hb2, gamma2, beta2, wlin2, blin2, wrel2, brel2, wroot2, w_out1, b_out1, w_out2, b_out2):
    raise NotImplementedError("write your pallas kernel here")


# speedup vs baseline: 14.4972x; 14.4972x over previous
import functools

import jax
import jax.numpy as jnp
from jax import lax
from jax.experimental import pallas as pl
from jax.experimental.pallas import tpu as pltpu
from jax.experimental.pallas import tpu_sc as plsc

_N = 10000
_M = 6000
_E = 320000
_H = 128
_G = 16
_SLOPE = 0.01
_EPS = 1e-5

_NCORE = 2
_NSUB = 16
_NW = _NCORE * _NSUB

_K = 128
_RPW = 80
_RTOT = _RPW * _NW
_EPAD = _RTOT * _K

_RPW_S = 8
_RTOT_S = _RPW_S * _NW
_EPAD_S = _RTOT_S * _K

_PAD = 1024
_SPAD = 11264
_ACC_M = 7168
_ACC_N = 11264
_CR = 64


def _mesh():
  return plsc.VectorSubcoreMesh(
      core_axis_name="c", subcore_axis_name="s",
      num_cores=_NCORE, num_subcores=_NSUB)


def _make_featpass(tacc, rpw):
  rows_per_tile = tacc // _NSUB
  nchunk = rows_per_tile // _CR

  @functools.partial(
      pl.kernel,
      out_type=jax.ShapeDtypeStruct((_NCORE, tacc, _H), jnp.float32),
      mesh=_mesh(),
      scratch_types=[
          pltpu.VMEM((rpw, _K), jnp.int32),
          pltpu.VMEM((rpw, _K), jnp.int32),
          pltpu.VMEM((_K, _H), jnp.float32),
          pltpu.VMEM_SHARED((tacc, _H), jnp.float32),
          pltpu.SemaphoreType.DMA,
      ],
  )
  def fp(src_hbm, idxin_hbm, idxout_hbm, zeros_hbm, out_hbm,
         idxi_v, idxo_v, rows_v, acc_sh, sem):
    c = lax.axis_index("c")
    s = lax.axis_index("s")
    w = s * _NCORE + c
    base0 = s * rows_per_tile

    pltpu.sync_copy(zeros_hbm, rows_v.at[pl.ds(0, _CR)])

    @pl.loop(0, nchunk)
    def _zero(i):
      pltpu.sync_copy(rows_v.at[pl.ds(0, _CR)], acc_sh.at[pl.ds(base0 + i * _CR, _CR)])

    pltpu.sync_copy(idxin_hbm.at[pl.ds(w * rpw, rpw)], idxi_v)
    pltpu.sync_copy(idxout_hbm.at[pl.ds(w * rpw, rpw)], idxo_v)
    plsc.subcore_barrier()

    @pl.loop(0, rpw)
    def _edge(j):
      pltpu.async_copy(src_hbm.at[idxi_v.at[j]], rows_v, sem).wait()
      pltpu.sync_copy(rows_v, acc_sh.at[idxo_v.at[j]], add=True)

    plsc.subcore_barrier()

    @pl.loop(0, nchunk)
    def _out(i):
      off = base0 + i * _CR
      pltpu.sync_copy(acc_sh.at[pl.ds(off, _CR)], rows_v.at[pl.ds(0, _CR)])
      pltpu.sync_copy(rows_v.at[pl.ds(0, _CR)], out_hbm.at[c, pl.ds(off, _CR)])

  return fp


def _make_scalarpass(t1, t2):
  d1 = t1 // _NSUB
  d2 = t2 // _NSUB

  @functools.partial(
      pl.kernel,
      out_type=(jax.ShapeDtypeStruct((_NCORE * t1,), jnp.float32),
                jax.ShapeDtypeStruct((_NCORE * t2,), jnp.float32)),
      mesh=_mesh(),
      compiler_params=pltpu.CompilerParams(needs_layout_passes=False),
      scratch_types=[
          pltpu.VMEM((_RPW, _K), jnp.int32),
          pltpu.VMEM((_RPW, _K), jnp.int32),
          pltpu.VMEM((_SPAD,), jnp.float32),
          pltpu.VMEM((_K,), jnp.float32),
          pltpu.VMEM((_K,), jnp.float32),
          pltpu.VMEM((max(t1, t2) // _NSUB,), jnp.float32),
          pltpu.VMEM_SHARED((t1,), jnp.float32),
          pltpu.VMEM_SHARED((t2,), jnp.float32),
      ],
  )
  def sp(row_hbm, col_hbm, hewt_hbm, zeros1_hbm, dd_hbm, bd_hbm,
         row_v, col_v, hewt_v, val_v, one_v, stg_v, dd_sh, bd_sh):
    c = lax.axis_index("c")
    s = lax.axis_index("s")
    w = s * _NCORE + c

    pltpu.sync_copy(zeros1_hbm.at[pl.ds(0, d1)], stg_v.at[pl.ds(0, d1)])
    pltpu.sync_copy(stg_v.at[pl.ds(0, d1)], dd_sh.at[pl.ds(s * d1, d1)])
    pltpu.sync_copy(stg_v.at[pl.ds(0, d2)], bd_sh.at[pl.ds(s * d2, d2)])
    pltpu.sync_copy(hewt_hbm, hewt_v)
    pltpu.sync_copy(row_hbm.at[pl.ds(w * _RPW, _RPW)], row_v)
    pltpu.sync_copy(col_hbm.at[pl.ds(w * _RPW, _RPW)], col_v)
    for i in range(_K // 16):
      one_v[pl.ds(i * 16, 16)] = jnp.ones((16,), jnp.float32)
    plsc.subcore_barrier()

    @pl.loop(0, _RPW)
    def _edge(j):
      for i in range(_K // 16):
        cv = col_v[j, pl.ds(i * 16, 16)]
        val_v[pl.ds(i * 16, 16)] = plsc.load_gather(hewt_v, [cv])
      pltpu.sync_copy(val_v, dd_sh.at[row_v.at[j]], add=True)
      pltpu.sync_copy(one_v, bd_sh.at[col_v.at[j]], add=True)

    plsc.subcore_barrier()
    pltpu.sync_copy(dd_sh.at[pl.ds(s * d1, d1)], stg_v.at[pl.ds(0, d1)])
    pltpu.sync_copy(stg_v.at[pl.ds(0, d1)], dd_hbm.at[pl.ds(c * t1 + s * d1, d1)])
    pltpu.sync_copy(bd_sh.at[pl.ds(s * d2, d2)], stg_v.at[pl.ds(0, d2)])
    pltpu.sync_copy(stg_v.at[pl.ds(0, d2)], bd_hbm.at[pl.ds(c * t2 + s * d2, d2)])

  return sp


def _make_filterpass():
  t = _ACC_N
  d = t // _NSUB

  @functools.partial(
      pl.kernel,
      out_type=(jax.ShapeDtypeStruct((_RTOT, _K), jnp.int32),
                jax.ShapeDtypeStruct((_RTOT, _K), jnp.int32),
                jax.ShapeDtypeStruct((_NCORE * t,), jnp.float32),
                jax.ShapeDtypeStruct((_NCORE * t,), jnp.float32)),
      mesh=_mesh(),
      compiler_params=pltpu.CompilerParams(needs_layout_passes=False),
      scratch_types=[
          pltpu.VMEM((_RPW, _K), jnp.int32),
          pltpu.VMEM((_RPW, _K), jnp.int32),
          pltpu.VMEM((_RPW, _K), jnp.int32),
          pltpu.VMEM((_RPW, _K), jnp.int32),
          pltpu.VMEM((_SPAD,), jnp.int32),
          pltpu.VMEM((_SPAD,), jnp.float32),
          pltpu.VMEM((_K,), jnp.float32),
          pltpu.VMEM((_K,), jnp.float32),
          pltpu.VMEM((d,), jnp.float32),
          pltpu.VMEM_SHARED((t,), jnp.float32),
          pltpu.VMEM_SHARED((t,), jnp.float32),
      ],
  )
  def fk(row_hbm, col_hbm, tbl_hbm, hewt_hbm, zeros1_hbm,
         nr_hbm, nc_hbm, dd_hbm, bd_hbm,
         row_v, col_v, nr_v, nc_v, tbl_v, hewt_v, val_v, one_v, stg_v,
         dd_sh, bd_sh):
    c = lax.axis_index("c")
    s = lax.axis_index("s")
    w = s * _NCORE + c

    pltpu.sync_copy(zeros1_hbm.at[pl.ds(0, d)], stg_v)
    pltpu.sync_copy(stg_v, dd_sh.at[pl.ds(s * d, d)])
    pltpu.sync_copy(stg_v, bd_sh.at[pl.ds(s * d, d)])
    pltpu.sync_copy(tbl_hbm, tbl_v)
    pltpu.sync_copy(hewt_hbm, hewt_v)
    pltpu.sync_copy(row_hbm.at[pl.ds(w * _RPW, _RPW)], row_v)
    pltpu.sync_copy(col_hbm.at[pl.ds(w * _RPW, _RPW)], col_v)
    for i in range(_K // 16):
      one_v[pl.ds(i * 16, 16)] = jnp.ones((16,), jnp.float32)
    lanes = lax.iota(jnp.int32, 16)
    plsc.subcore_barrier()

    @pl.loop(0, _RPW)
    def _edge(j):
      for i in range(_K // 16):
        rv = row_v[j, pl.ds(i * 16, 16)]
        cv = col_v[j, pl.ds(i * 16, 16)]
        tr = plsc.load_gather(tbl_v, [rv])
        tc = plsc.load_gather(tbl_v, [cv])
        keep = (tr >= 0) & (tc >= 0)
        gbase = (w * _RPW + j) * _K + i * 16
        sent = _N + ((gbase + lanes) & (_PAD - 1))
        nr = jnp.where(keep, tr, sent)
        nc = jnp.where(keep, tc, sent)
        nr_v[j, pl.ds(i * 16, 16)] = nr
        nc_v[j, pl.ds(i * 16, 16)] = nc
        val_v[pl.ds(i * 16, 16)] = plsc.load_gather(hewt_v, [nc])
      pltpu.sync_copy(val_v, dd_sh.at[nr_v.at[j]], add=True)
      pltpu.sync_copy(one_v, bd_sh.at[nc_v.at[j]], add=True)

    pltpu.sync_copy(nr_v, nr_hbm.at[pl.ds(w * _RPW, _RPW)])
    pltpu.sync_copy(nc_v, nc_hbm.at[pl.ds(w * _RPW, _RPW)])
    plsc.subcore_barrier()
    pltpu.sync_copy(dd_sh.at[pl.ds(s * d, d)], stg_v)
    pltpu.sync_copy(stg_v, dd_hbm.at[pl.ds(c * t + s * d, d)])
    pltpu.sync_copy(bd_sh.at[pl.ds(s * d, d)], stg_v)
    pltpu.sync_copy(stg_v, bd_hbm.at[pl.ds(c * t + s * d, d)])

  return fk


_sc_cache = {}


def _feat_m(*args):
  if "fm" not in _sc_cache:
    _sc_cache["fm"] = _make_featpass(_ACC_M, _RPW)
  return _sc_cache["fm"](*args)


def _feat_n(*args):
  if "fn" not in _sc_cache:
    _sc_cache["fn"] = _make_featpass(_ACC_N, _RPW)
  return _sc_cache["fn"](*args)


def _feat_s(*args):
  if "fs" not in _sc_cache:
    _sc_cache["fs"] = _make_featpass(_ACC_N, _RPW_S)
  return _sc_cache["fs"](*args)


def _scal_1(*args):
  if "s1" not in _sc_cache:
    _sc_cache["s1"] = _make_scalarpass(_ACC_N, _ACC_M)
  return _sc_cache["s1"](*args)


def _filter2(*args):
  if "f2" not in _sc_cache:
    _sc_cache["f2"] = _make_filterpass()
  return _sc_cache["f2"](*args)


_BR = 1000


def _dense2(a, w1, w2, b2):
  def body(a_ref, w1_ref, w2_ref, b2_ref, o1_ref, o2_ref):
    ab = a_ref[...]
    o1_ref[...] = jnp.dot(ab, w1_ref[...], preferred_element_type=jnp.float32)
    t = jnp.dot(ab, w2_ref[...], preferred_element_type=jnp.float32) + b2_ref[...]
    o2_ref[...] = jnp.where(t > 0, t, _SLOPE * t)

  grid = (_N // _BR,)
  return pl.pallas_call(
      body,
      grid=grid,
      in_specs=[
          pl.BlockSpec((_BR, _H), lambda i: (i, 0)),
          pl.BlockSpec((_H, _H), lambda i: (0, 0)),
          pl.BlockSpec((_H, _H), lambda i: (0, 0)),
          pl.BlockSpec((1, _H), lambda i: (0, 0)),
      ],
      out_specs=[
          pl.BlockSpec((_BR, _H), lambda i: (i, 0)),
          pl.BlockSpec((_BR, _H), lambda i: (i, 0)),
      ],
      out_shape=[
          jax.ShapeDtypeStruct((_N, _H), jnp.float32),
          jax.ShapeDtypeStruct((_N, _H), jnp.float32),
      ],
  )(a, w1, w2, b2.reshape(1, _H))


def _score_tc(agg, z, wrel, wroot, brel):
  wr = jnp.zeros((_H, _H), jnp.float32).at[:, 0].set(wrel.reshape(-1))
  wo = jnp.zeros((_H, _H), jnp.float32).at[:, 0].set(wroot.reshape(-1))

  def body(a_ref, z_ref, wr_ref, wo_ref, b_ref, o_ref):
    t = (jnp.dot(a_ref[...], wr_ref[...], preferred_element_type=jnp.float32)
         + jnp.dot(z_ref[...], wo_ref[...], preferred_element_type=jnp.float32)
         + b_ref[0, 0])
    o_ref[...] = jnp.tanh(t)

  out = pl.pallas_call(
      body,
      grid=(_N // _BR,),
      in_specs=[
          pl.BlockSpec((_BR, _H), lambda i: (i, 0)),
          pl.BlockSpec((_BR, _H), lambda i: (i, 0)),
          pl.BlockSpec((_H, _H), lambda i: (0, 0)),
          pl.BlockSpec((_H, _H), lambda i: (0, 0)),
          pl.BlockSpec((1, 1), lambda i: (0, 0), memory_space=pltpu.SMEM),
      ],
      out_specs=pl.BlockSpec((_BR, _H), lambda i: (i, 0)),
      out_shape=jax.ShapeDtypeStruct((_N, _H), jnp.float32),
  )(agg, z, wr, wo, brel.reshape(1, 1))
  return out[:, 0]


def _head_tc(o, w1, b1, w2, b2):
  w1p = jnp.zeros((_H, _H), jnp.float32).at[:, :w1.shape[1]].set(w1)
  b1p = jnp.zeros((1, _H), jnp.float32).at[0, :b1.shape[0]].set(b1)
  w2p = jnp.zeros((_H, _H), jnp.float32).at[:w2.shape[0], :w2.shape[1]].set(w2)
  b2p = jnp.zeros((1, _H), jnp.float32).at[0, :b2.shape[0]].set(b2)

  def body(o_ref, w1_ref, b1_ref, w2_ref, b2_ref, out_ref):
    t = jnp.dot(o_ref[...], w1_ref[...], preferred_element_type=jnp.float32) + b1_ref[...]
    h = jnp.where(t > 0, t, _SLOPE * t)
    out_ref[...] = jnp.dot(h, w2_ref[...], preferred_element_type=jnp.float32) + b2_ref[...]

  out = pl.pallas_call(
      body,
      out_shape=jax.ShapeDtypeStruct((_G, _H), jnp.float32),
  )(o, w1p, b1p, w2p, b2p)
  return out[:, :w2.shape[1]]


def _lrelu(v):
  return jnp.where(v > 0, v, _SLOPE * v)


def _pad_edges(idx, base):
  pad = base + (jnp.arange(_EPAD - _E, dtype=jnp.int32) & (_PAD - 1))
  return jnp.concatenate([idx, pad]).reshape(_RTOT, _K)


def _pad_rows(a):
  return jnp.pad(a, ((0, _SPAD - a.shape[0]), (0, 0)))


def _onehot_readout(z, b, valid_w, cnt):
  oh = (b[None, :] == jnp.arange(_G, dtype=b.dtype)[:, None]).astype(jnp.float32)
  oh = oh * valid_w[None, :]
  mean = (oh @ z) / jnp.maximum(cnt.astype(jnp.float32), 1.0)[:, None]
  bm = jnp.where(valid_w > 0, b, _G)
  mx = jax.ops.segment_max(z, bm, num_segments=_G)
  mx = jnp.where(jnp.isfinite(mx), mx, 0.0)
  return mean + mx


def _topk_masks(score, seg, counts):
  n = score.shape[0]
  iota = jnp.arange(n)
  order = jnp.lexsort((iota, -score, seg))
  sseg = seg[order]
  starts = jnp.cumsum(counts) - counts
  safe = jnp.minimum(sseg, _G - 1)
  k = jnp.ceil(0.5 * counts.astype(score.dtype)).astype(counts.dtype)
  pos = iota - starts[safe]
  valid = (sseg < _G) & (pos < k[safe])
  return order, valid, k


def _sorted_counts(seg):
  edges = jnp.searchsorted(seg, jnp.arange(_G + 1, dtype=seg.dtype), side="left")
  return (edges[1:] - edges[:-1]).astype(jnp.int32)


def kernel(x, hyperedge_index, hyperedge_weight, batch,
           theta1, hb1, gamma1, beta1, wlin1, blin1, wrel1, brel1, wroot1,
           theta2, hb2, gamma2, beta2, wlin2, blin2, wrel2, brel2, wroot2,
           w_out1, b_out1, w_out2, b_out2):
  row = hyperedge_index[0].astype(jnp.int32)
  col = hyperedge_index[1].astype(jnp.int32)
  zeros2d = jnp.zeros((_CR, _H), jnp.float32)
  zeros1d = jnp.zeros((_SPAD,), jnp.float32)

  row_g = _pad_edges(row, _N)
  col_g = _pad_edges(col, _N)
  col_sc = _pad_edges(col, _M)

  xt, xlin = _dense2(x, theta1, wlin1, blin1)

  he_p = _feat_m(_pad_rows(xt), row_g, col_sc, zeros2d)
  he_sum = he_p[0, :_M] + he_p[1, :_M]
  hewt1 = jnp.zeros((_SPAD,), jnp.float32).at[:_M].set(hyperedge_weight)
  dd_p, bd_p = _scal_1(row_g, col_sc, hewt1, zeros1d)
  dd_p = dd_p.reshape(_NCORE, _ACC_N)
  bd_p = bd_p.reshape(_NCORE, _ACC_M)
  dd = dd_p[0, :_N] + dd_p[1, :_N]
  bd = bd_p[0, :_M] + bd_p[1, :_M]
  dinv = jnp.where(dd > 0, 1.0 / dd, 0.0)
  binv = jnp.where(bd > 0, 1.0 / bd, 0.0)
  he_scaled = he_sum * (binv * hyperedge_weight)[:, None]
  out_p = _feat_n(_pad_rows(he_scaled), col_g, row_g, zeros2d)
  conv1 = (out_p[0, :_N] + out_p[1, :_N]) * dinv[:, None] + hb1

  mu = conv1.mean(0)
  var = conv1.var(0)
  z = _lrelu((conv1 - mu) / jnp.sqrt(var + _EPS) * gamma1 + beta1) + xlin

  agg_p = _feat_m(_pad_rows(z), row_g, col_sc, zeros2d)
  agg = jnp.pad(agg_p[0, :_M] + agg_p[1, :_M], ((0, _N - _M), (0, 0)))
  s1 = _score_tc(agg, z, wrel1, wroot1, brel1)

  counts1 = _sorted_counts(batch)
  order1, valid1, k1g = _topk_masks(s1, batch, counts1)
  compact1 = jnp.cumsum(valid1) - 1
  iota_n = jnp.arange(_N, dtype=jnp.int32)
  spread_n = _N + (iota_n & (_PAD - 1))
  dest1 = jnp.where(valid1, compact1.astype(jnp.int32), spread_n)
  tbl = jnp.full((_SPAD,), -1, jnp.int32).at[order1].set(
      jnp.where(valid1, compact1.astype(jnp.int32), -1))
  k1 = jnp.sum(valid1)
  kcum1 = jnp.cumsum(k1g)
  b1 = jnp.searchsorted(kcum1, iota_n, side="right").astype(batch.dtype)

  pad_s = _N + (jnp.arange(_EPAD_S - _N, dtype=jnp.int32) & (_PAD - 1))
  oidx = jnp.concatenate([order1.astype(jnp.int32), pad_s]).reshape(_RTOT_S, _K)
  didx = jnp.concatenate([dest1, pad_s]).reshape(_RTOT_S, _K)
  z1_p = _feat_s(_pad_rows(z * s1[:, None]), oidx, didx, zeros2d)
  z1 = z1_p[0, :_N] + z1_p[1, :_N]

  rmask = (iota_n < k1).astype(jnp.float32)
  out_r = _onehot_readout(z1, b1, rmask, k1g)

  hew_ext = jnp.concatenate(
      [hyperedge_weight,
       jnp.full((_N - _M,), hyperedge_weight[_M - 1], jnp.float32),
       jnp.zeros((_SPAD - _N,), jnp.float32)])
  nr2, nc2, dd2_p, bd2_p = _filter2(row_g, col_g, tbl, hew_ext, zeros1d)
  dd2_p = dd2_p.reshape(_NCORE, _ACC_N)
  bd2_p = bd2_p.reshape(_NCORE, _ACC_N)
  dd2 = dd2_p[0, :_N] + dd2_p[1, :_N]
  bd2 = bd2_p[0, :_N] + bd2_p[1, :_N]
  dinv2 = jnp.where(dd2 > 0, 1.0 / dd2, 0.0)
  binv2 = jnp.where(bd2 > 0, 1.0 / bd2, 0.0)

  xt2, xlin2 = _dense2(z1, theta2, wlin2, blin2)
  he2_p = _feat_n(_pad_rows(xt2), nr2, nc2, zeros2d)
  he2 = (he2_p[0, :_N] + he2_p[1, :_N]) * (binv2 * hew_ext[:_N])[:, None]
  out2_p = _feat_n(_pad_rows(he2), nc2, nr2, zeros2d)
  conv2 = (out2_p[0, :_N] + out2_p[1, :_N]) * dinv2[:, None] + hb2

  cmax = jnp.maximum(k1.astype(jnp.float32), 1.0)
  mu2 = (conv2 * rmask[:, None]).sum(0) / cmax
  var2 = (((conv2 - mu2) * rmask[:, None]) ** 2).sum(0) / cmax
  z2l = (_lrelu((conv2 - mu2) / jnp.sqrt(var2 + _EPS) * gamma2 + beta2)
         + xlin2) * rmask[:, None]

  agg2_p = _feat_n(_pad_rows(z2l), nr2, nc2, zeros2d)
  agg2 = agg2_p[0, :_N] + agg2_p[1, :_N]
  s2 = _score_tc(agg2, z2l, wrel2, wroot2, brel2)

  counts2 = _sorted_counts(b1)
  order2, valid2, k2g = _topk_masks(s2, b1, counts2)
  vslot = jnp.zeros((_N,), jnp.bool_).at[order2].set(valid2)
  w2v = jnp.where(vslot, s2, 0.0)
  zw2 = z2l * w2v[:, None]
  out_r = out_r + _onehot_readout(zw2, b1, vslot.astype(jnp.float32), k2g)

  return _head_tc(out_r, w_out1, b_out1, w_out2, b_out2)


# speedup vs baseline: 16.6298x; 1.1471x over previous
import functools

import jax
import jax.numpy as jnp
from jax import lax
from jax.experimental import pallas as pl
from jax.experimental.pallas import tpu as pltpu
from jax.experimental.pallas import tpu_sc as plsc

_N = 10000
_M = 6000
_E = 320000
_H = 128
_G = 16
_SLOPE = 0.01
_EPS = 1e-5

_NCORE = 2
_NSUB = 16
_NW = _NCORE * _NSUB

_K = 128
_RPW = 80
_RTOT = _RPW * _NW
_EPAD = _RTOT * _K

_RPW_S = 8
_RTOT_S = _RPW_S * _NW
_EPAD_S = _RTOT_S * _K

_PAD = 1024
_SPAD = 11264
_ACC_M = 7168
_ACC_N = 11264
_CR = 64


def _mesh():
  return plsc.VectorSubcoreMesh(
      core_axis_name="c", subcore_axis_name="s",
      num_cores=_NCORE, num_subcores=_NSUB)


def _make_featpass(tacc, rpw):
  rows_per_tile = tacc // _NSUB
  nchunk = rows_per_tile // _CR

  @functools.partial(
      pl.kernel,
      out_type=jax.ShapeDtypeStruct((_NCORE, tacc, _H), jnp.float32),
      mesh=_mesh(),
      scratch_types=[
          pltpu.VMEM((8, _K), jnp.int32),
          pltpu.VMEM((8, _K), jnp.int32),
          pltpu.VMEM((_K, _H), jnp.float32),
          pltpu.VMEM((_K, _H), jnp.float32),
          pltpu.VMEM_SHARED((tacc, _H), jnp.float32),
          pltpu.SemaphoreType.DMA,
          pltpu.SemaphoreType.DMA,
          pltpu.SemaphoreType.DMA,
          pltpu.SemaphoreType.DMA,
      ],
  )
  def fp(src_hbm, idxin_hbm, idxout_hbm, zeros_hbm, out_hbm,
         idxi_v, idxo_v, rows_v, rows2_v, acc_sh, ga, gb, sa, sb):
    c = lax.axis_index("c")
    s = lax.axis_index("s")
    w = s * _NCORE + c
    base0 = s * rows_per_tile

    pltpu.sync_copy(zeros_hbm, rows_v.at[pl.ds(0, _CR)])

    @pl.loop(0, nchunk)
    def _zero(i):
      pltpu.sync_copy(rows_v.at[pl.ds(0, _CR)], acc_sh.at[pl.ds(base0 + i * _CR, _CR)])

    plsc.subcore_barrier()

    def _drain(buf, sem):
      pltpu.make_async_copy(src_hbm.at[pl.ds(0, _K)], buf, sem).wait()

    @pl.loop(0, rpw // 8)
    def _grp(g):
      gbase = w * rpw + g * 8
      pltpu.sync_copy(idxin_hbm.at[pl.ds(gbase, 8)], idxi_v)
      pltpu.sync_copy(idxout_hbm.at[pl.ds(gbase, 8)], idxo_v)
      pltpu.async_copy(src_hbm.at[idxi_v.at[0]], rows_v, ga)
      for jj in range(4):
        j = jj * 2
        _drain(rows_v, ga)
        pltpu.async_copy(src_hbm.at[idxi_v.at[j + 1]], rows2_v, gb)
        pltpu.async_copy(rows_v, acc_sh.at[idxo_v.at[j]], sa, add=True)
        _drain(rows2_v, gb)
        _drain(rows_v, sa)
        if j + 2 < 8:
          pltpu.async_copy(src_hbm.at[idxi_v.at[j + 2]], rows_v, ga)
        pltpu.async_copy(rows2_v, acc_sh.at[idxo_v.at[j + 1]], sb, add=True)
        _drain(rows2_v, sb)

    plsc.subcore_barrier()

    @pl.loop(0, nchunk)
    def _out(i):
      off = base0 + i * _CR
      pltpu.sync_copy(acc_sh.at[pl.ds(off, _CR)], rows_v.at[pl.ds(0, _CR)])
      pltpu.sync_copy(rows_v.at[pl.ds(0, _CR)], out_hbm.at[c, pl.ds(off, _CR)])

  return fp


def _make_scalarpass(t1, t2):
  d1 = t1 // _NSUB
  d2 = t2 // _NSUB

  @functools.partial(
      pl.kernel,
      out_type=(jax.ShapeDtypeStruct((_NCORE * t1,), jnp.float32),
                jax.ShapeDtypeStruct((_NCORE * t2,), jnp.float32)),
      mesh=_mesh(),
      compiler_params=pltpu.CompilerParams(needs_layout_passes=False),
      scratch_types=[
          pltpu.VMEM((_RPW, _K), jnp.int32),
          pltpu.VMEM((_RPW, _K), jnp.int32),
          pltpu.VMEM((_SPAD,), jnp.float32),
          pltpu.VMEM((_RPW, _K), jnp.float32),
          pltpu.VMEM((_K,), jnp.float32),
          pltpu.VMEM((max(t1, t2) // _NSUB,), jnp.float32),
          pltpu.VMEM_SHARED((t1,), jnp.float32),
          pltpu.VMEM_SHARED((t2,), jnp.float32),
          pltpu.SemaphoreType.DMA,
          pltpu.SemaphoreType.DMA,
      ],
  )
  def sp(row_hbm, col_hbm, hewt_hbm, zeros1_hbm, dd_hbm, bd_hbm,
         row_v, col_v, hewt_v, val_f, one_v, stg_v, dd_sh, bd_sh, sd, sb):
    c = lax.axis_index("c")
    s = lax.axis_index("s")
    w = s * _NCORE + c

    pltpu.sync_copy(zeros1_hbm.at[pl.ds(0, d1)], stg_v.at[pl.ds(0, d1)])
    pltpu.sync_copy(stg_v.at[pl.ds(0, d1)], dd_sh.at[pl.ds(s * d1, d1)])
    pltpu.sync_copy(stg_v.at[pl.ds(0, d2)], bd_sh.at[pl.ds(s * d2, d2)])
    pltpu.sync_copy(hewt_hbm, hewt_v)
    pltpu.sync_copy(row_hbm.at[pl.ds(w * _RPW, _RPW)], row_v)
    pltpu.sync_copy(col_hbm.at[pl.ds(w * _RPW, _RPW)], col_v)
    for i in range(_K // 16):
      one_v[pl.ds(i * 16, 16)] = jnp.ones((16,), jnp.float32)
    plsc.subcore_barrier()

    @pl.loop(0, _RPW)
    def _vals(j):
      for i in range(_K // 16):
        cv = col_v[j, pl.ds(i * 16, 16)]
        val_f[j, pl.ds(i * 16, 16)] = plsc.load_gather(hewt_v, [cv])

    def _drain1(sem):
      pltpu.make_async_copy(zeros1_hbm.at[pl.ds(0, _K)], one_v, sem).wait()

    @pl.loop(0, _RPW // 8)
    def _grp(g):
      for t in range(8):
        j = g * 8 + t
        pltpu.async_copy(val_f.at[j], dd_sh.at[row_v.at[j]], sd, add=True)
        pltpu.async_copy(one_v, bd_sh.at[col_v.at[j]], sb, add=True)
      for t in range(8):
        _drain1(sd)
        _drain1(sb)

    plsc.subcore_barrier()
    pltpu.sync_copy(dd_sh.at[pl.ds(s * d1, d1)], stg_v.at[pl.ds(0, d1)])
    pltpu.sync_copy(stg_v.at[pl.ds(0, d1)], dd_hbm.at[pl.ds(c * t1 + s * d1, d1)])
    pltpu.sync_copy(bd_sh.at[pl.ds(s * d2, d2)], stg_v.at[pl.ds(0, d2)])
    pltpu.sync_copy(stg_v.at[pl.ds(0, d2)], bd_hbm.at[pl.ds(c * t2 + s * d2, d2)])

  return sp


def _make_filterpass():
  t = _ACC_N
  d = t // _NSUB

  @functools.partial(
      pl.kernel,
      out_type=(jax.ShapeDtypeStruct((_RTOT, _K), jnp.int32),
                jax.ShapeDtypeStruct((_RTOT, _K), jnp.int32),
                jax.ShapeDtypeStruct((_NCORE * t,), jnp.float32),
                jax.ShapeDtypeStruct((_NCORE * t,), jnp.float32)),
      mesh=_mesh(),
      compiler_params=pltpu.CompilerParams(needs_layout_passes=False),
      scratch_types=[
          pltpu.VMEM((_RPW, _K), jnp.int32),
          pltpu.VMEM((_RPW, _K), jnp.int32),
          pltpu.VMEM((_RPW, _K), jnp.int32),
          pltpu.VMEM((_RPW, _K), jnp.int32),
          pltpu.VMEM((_SPAD,), jnp.int32),
          pltpu.VMEM((_SPAD,), jnp.float32),
          pltpu.VMEM((_RPW, _K), jnp.float32),
          pltpu.VMEM((_K,), jnp.float32),
          pltpu.VMEM((d,), jnp.float32),
          pltpu.VMEM_SHARED((t,), jnp.float32),
          pltpu.VMEM_SHARED((t,), jnp.float32),
          pltpu.SemaphoreType.DMA,
          pltpu.SemaphoreType.DMA,
      ],
  )
  def fk(row_hbm, col_hbm, tbl_hbm, hewt_hbm, zeros1_hbm,
         nr_hbm, nc_hbm, dd_hbm, bd_hbm,
         row_v, col_v, nr_v, nc_v, tbl_v, hewt_v, val_f, one_v, stg_v,
         dd_sh, bd_sh, sd, sb):
    c = lax.axis_index("c")
    s = lax.axis_index("s")
    w = s * _NCORE + c

    pltpu.sync_copy(zeros1_hbm.at[pl.ds(0, d)], stg_v)
    pltpu.sync_copy(stg_v, dd_sh.at[pl.ds(s * d, d)])
    pltpu.sync_copy(stg_v, bd_sh.at[pl.ds(s * d, d)])
    pltpu.sync_copy(tbl_hbm, tbl_v)
    pltpu.sync_copy(hewt_hbm, hewt_v)
    pltpu.sync_copy(row_hbm.at[pl.ds(w * _RPW, _RPW)], row_v)
    pltpu.sync_copy(col_hbm.at[pl.ds(w * _RPW, _RPW)], col_v)
    for i in range(_K // 16):
      one_v[pl.ds(i * 16, 16)] = jnp.ones((16,), jnp.float32)
    lanes = lax.iota(jnp.int32, 16)
    plsc.subcore_barrier()

    @pl.loop(0, _RPW)
    def _edge(j):
      for i in range(_K // 16):
        rv = row_v[j, pl.ds(i * 16, 16)]
        cv = col_v[j, pl.ds(i * 16, 16)]
        tr = plsc.load_gather(tbl_v, [rv])
        tc = plsc.load_gather(tbl_v, [cv])
        keep = (tr >= 0) & (tc >= 0)
        gbase = (w * _RPW + j) * _K + i * 16
        sent = _N + ((gbase + lanes) & (_PAD - 1))
        nr = jnp.where(keep, tr, sent)
        nc = jnp.where(keep, tc, sent)
        nr_v[j, pl.ds(i * 16, 16)] = nr
        nc_v[j, pl.ds(i * 16, 16)] = nc
        val_f[j, pl.ds(i * 16, 16)] = plsc.load_gather(hewt_v, [nc])

    pltpu.sync_copy(nr_v, nr_hbm.at[pl.ds(w * _RPW, _RPW)])
    pltpu.sync_copy(nc_v, nc_hbm.at[pl.ds(w * _RPW, _RPW)])

    def _drain1(sem):
      pltpu.make_async_copy(zeros1_hbm.at[pl.ds(0, _K)], one_v, sem).wait()

    @pl.loop(0, _RPW // 8)
    def _grp(g):
      for u in range(8):
        j = g * 8 + u
        pltpu.async_copy(val_f.at[j], dd_sh.at[nr_v.at[j]], sd, add=True)
        pltpu.async_copy(one_v, bd_sh.at[nc_v.at[j]], sb, add=True)
      for u in range(8):
        _drain1(sd)
        _drain1(sb)

    plsc.subcore_barrier()
    pltpu.sync_copy(dd_sh.at[pl.ds(s * d, d)], stg_v)
    pltpu.sync_copy(stg_v, dd_hbm.at[pl.ds(c * t + s * d, d)])
    pltpu.sync_copy(bd_sh.at[pl.ds(s * d, d)], stg_v)
    pltpu.sync_copy(stg_v, bd_hbm.at[pl.ds(c * t + s * d, d)])

  return fk


_sc_cache = {}


def _feat_m(*args):
  if "fm" not in _sc_cache:
    _sc_cache["fm"] = _make_featpass(_ACC_M, _RPW)
  return _sc_cache["fm"](*args)


def _feat_n(*args):
  if "fn" not in _sc_cache:
    _sc_cache["fn"] = _make_featpass(_ACC_N, _RPW)
  return _sc_cache["fn"](*args)


def _feat_s(*args):
  if "fs" not in _sc_cache:
    _sc_cache["fs"] = _make_featpass(_ACC_N, _RPW_S)
  return _sc_cache["fs"](*args)


def _scal_1(*args):
  if "s1" not in _sc_cache:
    _sc_cache["s1"] = _make_scalarpass(_ACC_N, _ACC_M)
  return _sc_cache["s1"](*args)


def _filter2(*args):
  if "f2" not in _sc_cache:
    _sc_cache["f2"] = _make_filterpass()
  return _sc_cache["f2"](*args)


_BR = 1000


def _dense2(a, w1, w2, b2):
  def body(a_ref, w1_ref, w2_ref, b2_ref, o1_ref, o2_ref):
    ab = a_ref[...]
    o1_ref[...] = jnp.dot(ab, w1_ref[...], preferred_element_type=jnp.float32)
    t = jnp.dot(ab, w2_ref[...], preferred_element_type=jnp.float32) + b2_ref[...]
    o2_ref[...] = jnp.where(t > 0, t, _SLOPE * t)

  grid = (_N // _BR,)
  return pl.pallas_call(
      body,
      grid=grid,
      in_specs=[
          pl.BlockSpec((_BR, _H), lambda i: (i, 0)),
          pl.BlockSpec((_H, _H), lambda i: (0, 0)),
          pl.BlockSpec((_H, _H), lambda i: (0, 0)),
          pl.BlockSpec((1, _H), lambda i: (0, 0)),
      ],
      out_specs=[
          pl.BlockSpec((_BR, _H), lambda i: (i, 0)),
          pl.BlockSpec((_BR, _H), lambda i: (i, 0)),
      ],
      out_shape=[
          jax.ShapeDtypeStruct((_N, _H), jnp.float32),
          jax.ShapeDtypeStruct((_N, _H), jnp.float32),
      ],
  )(a, w1, w2, b2.reshape(1, _H))


def _score_tc(agg, z, wrel, wroot, brel):
  wr = jnp.zeros((_H, _H), jnp.float32).at[:, 0].set(wrel.reshape(-1))
  wo = jnp.zeros((_H, _H), jnp.float32).at[:, 0].set(wroot.reshape(-1))

  def body(a_ref, z_ref, wr_ref, wo_ref, b_ref, o_ref):
    t = (jnp.dot(a_ref[...], wr_ref[...], preferred_element_type=jnp.float32)
         + jnp.dot(z_ref[...], wo_ref[...], preferred_element_type=jnp.float32)
         + b_ref[0, 0])
    o_ref[...] = jnp.tanh(t)

  out = pl.pallas_call(
      body,
      grid=(_N // _BR,),
      in_specs=[
          pl.BlockSpec((_BR, _H), lambda i: (i, 0)),
          pl.BlockSpec((_BR, _H), lambda i: (i, 0)),
          pl.BlockSpec((_H, _H), lambda i: (0, 0)),
          pl.BlockSpec((_H, _H), lambda i: (0, 0)),
          pl.BlockSpec((1, 1), lambda i: (0, 0), memory_space=pltpu.SMEM),
      ],
      out_specs=pl.BlockSpec((_BR, _H), lambda i: (i, 0)),
      out_shape=jax.ShapeDtypeStruct((_N, _H), jnp.float32),
  )(agg, z, wr, wo, brel.reshape(1, 1))
  return out[:, 0]


def _head_tc(o, w1, b1, w2, b2):
  w1p = jnp.zeros((_H, _H), jnp.float32).at[:, :w1.shape[1]].set(w1)
  b1p = jnp.zeros((1, _H), jnp.float32).at[0, :b1.shape[0]].set(b1)
  w2p = jnp.zeros((_H, _H), jnp.float32).at[:w2.shape[0], :w2.shape[1]].set(w2)
  b2p = jnp.zeros((1, _H), jnp.float32).at[0, :b2.shape[0]].set(b2)

  def body(o_ref, w1_ref, b1_ref, w2_ref, b2_ref, out_ref):
    t = jnp.dot(o_ref[...], w1_ref[...], preferred_element_type=jnp.float32) + b1_ref[...]
    h = jnp.where(t > 0, t, _SLOPE * t)
    out_ref[...] = jnp.dot(h, w2_ref[...], preferred_element_type=jnp.float32) + b2_ref[...]

  out = pl.pallas_call(
      body,
      out_shape=jax.ShapeDtypeStruct((_G, _H), jnp.float32),
  )(o, w1p, b1p, w2p, b2p)
  return out[:, :w2.shape[1]]


def _lrelu(v):
  return jnp.where(v > 0, v, _SLOPE * v)


def _pad_edges(idx, base):
  pad = base + (jnp.arange(_EPAD - _E, dtype=jnp.int32) & (_PAD - 1))
  return jnp.concatenate([idx, pad]).reshape(_RTOT, _K)


def _pad_rows(a):
  return jnp.pad(a, ((0, _SPAD - a.shape[0]), (0, 0)))


def _onehot_readout(z, b, valid_w, cnt):
  oh = (b[None, :] == jnp.arange(_G, dtype=b.dtype)[:, None]).astype(jnp.float32)
  oh = oh * valid_w[None, :]
  mean = (oh @ z) / jnp.maximum(cnt.astype(jnp.float32), 1.0)[:, None]
  bm = jnp.where(valid_w > 0, b, _G)
  mx = jax.ops.segment_max(z, bm, num_segments=_G)
  mx = jnp.where(jnp.isfinite(mx), mx, 0.0)
  return mean + mx


def _topk_masks(score, seg, counts):
  n = score.shape[0]
  iota = jnp.arange(n)
  order = jnp.lexsort((iota, -score, seg))
  sseg = seg[order]
  starts = jnp.cumsum(counts) - counts
  safe = jnp.minimum(sseg, _G - 1)
  k = jnp.ceil(0.5 * counts.astype(score.dtype)).astype(counts.dtype)
  pos = iota - starts[safe]
  valid = (sseg < _G) & (pos < k[safe])
  return order, valid, k


def _sorted_counts(seg):
  edges = jnp.searchsorted(seg, jnp.arange(_G + 1, dtype=seg.dtype), side="left")
  return (edges[1:] - edges[:-1]).astype(jnp.int32)


def kernel(x, hyperedge_index, hyperedge_weight, batch,
           theta1, hb1, gamma1, beta1, wlin1, blin1, wrel1, brel1, wroot1,
           theta2, hb2, gamma2, beta2, wlin2, blin2, wrel2, brel2, wroot2,
           w_out1, b_out1, w_out2, b_out2):
  row = hyperedge_index[0].astype(jnp.int32)
  col = hyperedge_index[1].astype(jnp.int32)
  zeros2d = jnp.zeros((_CR, _H), jnp.float32)
  zeros1d = jnp.zeros((_SPAD,), jnp.float32)

  row_g = _pad_edges(row, _N)
  col_g = _pad_edges(col, _N)
  col_sc = _pad_edges(col, _M)

  xt, xlin = _dense2(x, theta1, wlin1, blin1)

  he_p = _feat_m(_pad_rows(xt), row_g, col_sc, zeros2d)
  he_sum = he_p[0, :_M] + he_p[1, :_M]
  hewt1 = jnp.zeros((_SPAD,), jnp.float32).at[:_M].set(hyperedge_weight)
  dd_p, bd_p = _scal_1(row_g, col_sc, hewt1, zeros1d)
  dd_p = dd_p.reshape(_NCORE, _ACC_N)
  bd_p = bd_p.reshape(_NCORE, _ACC_M)
  dd = dd_p[0, :_N] + dd_p[1, :_N]
  bd = bd_p[0, :_M] + bd_p[1, :_M]
  dinv = jnp.where(dd > 0, 1.0 / dd, 0.0)
  binv = jnp.where(bd > 0, 1.0 / bd, 0.0)
  he_scaled = he_sum * (binv * hyperedge_weight)[:, None]
  out_p = _feat_n(_pad_rows(he_scaled), col_g, row_g, zeros2d)
  conv1 = (out_p[0, :_N] + out_p[1, :_N]) * dinv[:, None] + hb1

  mu = conv1.mean(0)
  var = conv1.var(0)
  z = _lrelu((conv1 - mu) / jnp.sqrt(var + _EPS) * gamma1 + beta1) + xlin

  agg_p = _feat_m(_pad_rows(z), row_g, col_sc, zeros2d)
  agg = jnp.pad(agg_p[0, :_M] + agg_p[1, :_M], ((0, _N - _M), (0, 0)))
  s1 = _score_tc(agg, z, wrel1, wroot1, brel1)

  counts1 = _sorted_counts(batch)
  order1, valid1, k1g = _topk_masks(s1, batch, counts1)
  compact1 = jnp.cumsum(valid1) - 1
  iota_n = jnp.arange(_N, dtype=jnp.int32)
  spread_n = _N + (iota_n & (_PAD - 1))
  dest1 = jnp.where(valid1, compact1.astype(jnp.int32), spread_n)
  tbl = jnp.full((_SPAD,), -1, jnp.int32).at[order1].set(
      jnp.where(valid1, compact1.astype(jnp.int32), -1))
  k1 = jnp.sum(valid1)
  kcum1 = jnp.cumsum(k1g)
  b1 = jnp.searchsorted(kcum1, iota_n, side="right").astype(batch.dtype)

  pad_s = _N + (jnp.arange(_EPAD_S - _N, dtype=jnp.int32) & (_PAD - 1))
  oidx = jnp.concatenate([order1.astype(jnp.int32), pad_s]).reshape(_RTOT_S, _K)
  didx = jnp.concatenate([dest1, pad_s]).reshape(_RTOT_S, _K)
  z1_p = _feat_s(_pad_rows(z * s1[:, None]), oidx, didx, zeros2d)
  z1 = z1_p[0, :_N] + z1_p[1, :_N]

  rmask = (iota_n < k1).astype(jnp.float32)
  out_r = _onehot_readout(z1, b1, rmask, k1g)

  hew_ext = jnp.concatenate(
      [hyperedge_weight,
       jnp.full((_N - _M,), hyperedge_weight[_M - 1], jnp.float32),
       jnp.zeros((_SPAD - _N,), jnp.float32)])
  nr2, nc2, dd2_p, bd2_p = _filter2(row_g, col_g, tbl, hew_ext, zeros1d)
  dd2_p = dd2_p.reshape(_NCORE, _ACC_N)
  bd2_p = bd2_p.reshape(_NCORE, _ACC_N)
  dd2 = dd2_p[0, :_N] + dd2_p[1, :_N]
  bd2 = bd2_p[0, :_N] + bd2_p[1, :_N]
  dinv2 = jnp.where(dd2 > 0, 1.0 / dd2, 0.0)
  binv2 = jnp.where(bd2 > 0, 1.0 / bd2, 0.0)

  xt2, xlin2 = _dense2(z1, theta2, wlin2, blin2)
  he2_p = _feat_n(_pad_rows(xt2), nr2, nc2, zeros2d)
  he2 = (he2_p[0, :_N] + he2_p[1, :_N]) * (binv2 * hew_ext[:_N])[:, None]
  out2_p = _feat_n(_pad_rows(he2), nc2, nr2, zeros2d)
  conv2 = (out2_p[0, :_N] + out2_p[1, :_N]) * dinv2[:, None] + hb2

  cmax = jnp.maximum(k1.astype(jnp.float32), 1.0)
  mu2 = (conv2 * rmask[:, None]).sum(0) / cmax
  var2 = (((conv2 - mu2) * rmask[:, None]) ** 2).sum(0) / cmax
  z2l = (_lrelu((conv2 - mu2) / jnp.sqrt(var2 + _EPS) * gamma2 + beta2)
         + xlin2) * rmask[:, None]

  agg2_p = _feat_n(_pad_rows(z2l), nr2, nc2, zeros2d)
  agg2 = agg2_p[0, :_N] + agg2_p[1, :_N]
  s2 = _score_tc(agg2, z2l, wrel2, wroot2, brel2)

  counts2 = _sorted_counts(b1)
  order2, valid2, k2g = _topk_masks(s2, b1, counts2)
  vslot = jnp.zeros((_N,), jnp.bool_).at[order2].set(valid2)
  w2v = jnp.where(vslot, s2, 0.0)
  zw2 = z2l * w2v[:, None]
  out_r = out_r + _onehot_readout(zw2, b1, vslot.astype(jnp.float32), k2g)

  return _head_tc(out_r, w_out1, b_out1, w_out2, b_out2)


# speedup vs baseline: 20.3995x; 1.2267x over previous
import functools

import jax
import jax.numpy as jnp
from jax import lax
from jax.experimental import pallas as pl
from jax.experimental.pallas import tpu as pltpu
from jax.experimental.pallas import tpu_sc as plsc

_N = 10000
_M = 6000
_E = 320000
_H = 128
_G = 16
_SLOPE = 0.01
_EPS = 1e-5

_NCORE = 2
_NSUB = 16
_NW = _NCORE * _NSUB

_K = 128
_RPW = 80
_RTOT = _RPW * _NW
_EPAD = _RTOT * _K

_RPW_S = 8
_RTOT_S = _RPW_S * _NW
_EPAD_S = _RTOT_S * _K

_RC = 88

_PAD = 1024
_SPAD = 11264
_ACC_M = 7168
_ACC_N = 11264
_CR = 64


def _mesh():
  return plsc.VectorSubcoreMesh(
      core_axis_name="c", subcore_axis_name="s",
      num_cores=_NCORE, num_subcores=_NSUB)


def _make_featpass(tacc, rpw, dynamic=False):
  rows_per_tile = tacc // _NSUB
  nchunk = rows_per_tile // _CR
  in_cnt = ([jax.ShapeDtypeStruct((_NW * 16,), jnp.int32)] if dynamic else [])

  def make(body):
    return pl.kernel(
        body,
        out_type=jax.ShapeDtypeStruct((_NCORE, tacc, _H), jnp.float32),
        mesh=_mesh(),
        scratch_types=[
            pltpu.VMEM((8, _K), jnp.int32),
            pltpu.VMEM((8, _K), jnp.int32),
            pltpu.VMEM((16,), jnp.int32),
            pltpu.VMEM((_K, _H), jnp.float32),
            pltpu.VMEM((_K, _H), jnp.float32),
            pltpu.VMEM_SHARED((tacc, _H), jnp.float32),
            pltpu.SemaphoreType.DMA,
            pltpu.SemaphoreType.DMA,
            pltpu.SemaphoreType.DMA,
            pltpu.SemaphoreType.DMA,
        ])

  def fp_body(src_hbm, idxin_hbm, idxout_hbm, *rest):
    if dynamic:
      cnt_hbm = rest[0]
      rest = rest[1:]
    (zeros_hbm, out_hbm,
     idxi_v, idxo_v, cnt_v, rows_v, rows2_v, acc_sh, ga, gb, sa, sb) = rest
    c = lax.axis_index("c")
    s = lax.axis_index("s")
    w = s * _NCORE + c
    base0 = s * rows_per_tile

    pltpu.sync_copy(zeros_hbm, rows_v.at[pl.ds(0, _CR)])

    @pl.loop(0, nchunk)
    def _zero(i):
      pltpu.sync_copy(rows_v.at[pl.ds(0, _CR)], acc_sh.at[pl.ds(base0 + i * _CR, _CR)])

    plsc.subcore_barrier()

    def _drain(buf, sem):
      pltpu.make_async_copy(src_hbm.at[pl.ds(0, _K)], buf, sem).wait()

    if dynamic:
      pltpu.sync_copy(cnt_hbm.at[pl.ds(w * 16, 16)], cnt_v)
      ngrp = cnt_v[...][0]
    else:
      ngrp = rpw // 8

    @pl.loop(0, ngrp)
    def _grp(g):
      gbase = w * rpw + g * 8
      pltpu.sync_copy(idxin_hbm.at[pl.ds(gbase, 8)], idxi_v)
      pltpu.sync_copy(idxout_hbm.at[pl.ds(gbase, 8)], idxo_v)
      pltpu.async_copy(src_hbm.at[idxi_v.at[0]], rows_v, ga)
      for jj in range(4):
        j = jj * 2
        _drain(rows_v, ga)
        pltpu.async_copy(src_hbm.at[idxi_v.at[j + 1]], rows2_v, gb)
        pltpu.async_copy(rows_v, acc_sh.at[idxo_v.at[j]], sa, add=True)
        _drain(rows2_v, gb)
        _drain(rows_v, sa)
        if j + 2 < 8:
          pltpu.async_copy(src_hbm.at[idxi_v.at[j + 2]], rows_v, ga)
        pltpu.async_copy(rows2_v, acc_sh.at[idxo_v.at[j + 1]], sb, add=True)
        _drain(rows2_v, sb)

    plsc.subcore_barrier()

    @pl.loop(0, nchunk)
    def _out(i):
      off = base0 + i * _CR
      pltpu.sync_copy(acc_sh.at[pl.ds(off, _CR)], rows_v.at[pl.ds(0, _CR)])
      pltpu.sync_copy(rows_v.at[pl.ds(0, _CR)], out_hbm.at[c, pl.ds(off, _CR)])

  return make(fp_body)


def _make_scalarpass(t1, t2):
  d1 = t1 // _NSUB
  d2 = t2 // _NSUB

  @functools.partial(
      pl.kernel,
      out_type=(jax.ShapeDtypeStruct((_NCORE * t1,), jnp.float32),
                jax.ShapeDtypeStruct((_NCORE * t2,), jnp.float32)),
      mesh=_mesh(),
      compiler_params=pltpu.CompilerParams(needs_layout_passes=False),
      scratch_types=[
          pltpu.VMEM((_RPW, _K), jnp.int32),
          pltpu.VMEM((_RPW, _K), jnp.int32),
          pltpu.VMEM((_SPAD,), jnp.float32),
          pltpu.VMEM((_RPW, _K), jnp.float32),
          pltpu.VMEM((_K,), jnp.float32),
          pltpu.VMEM((max(t1, t2) // _NSUB,), jnp.float32),
          pltpu.VMEM_SHARED((t1,), jnp.float32),
          pltpu.VMEM_SHARED((t2,), jnp.float32),
          pltpu.SemaphoreType.DMA,
          pltpu.SemaphoreType.DMA,
      ],
  )
  def sp(row_hbm, col_hbm, hewt_hbm, zeros1_hbm, dd_hbm, bd_hbm,
         row_v, col_v, hewt_v, val_f, one_v, stg_v, dd_sh, bd_sh, sd, sb):
    c = lax.axis_index("c")
    s = lax.axis_index("s")
    w = s * _NCORE + c

    pltpu.sync_copy(zeros1_hbm.at[pl.ds(0, d1)], stg_v.at[pl.ds(0, d1)])
    pltpu.sync_copy(stg_v.at[pl.ds(0, d1)], dd_sh.at[pl.ds(s * d1, d1)])
    pltpu.sync_copy(stg_v.at[pl.ds(0, d2)], bd_sh.at[pl.ds(s * d2, d2)])
    pltpu.sync_copy(hewt_hbm, hewt_v)
    pltpu.sync_copy(row_hbm.at[pl.ds(w * _RPW, _RPW)], row_v)
    pltpu.sync_copy(col_hbm.at[pl.ds(w * _RPW, _RPW)], col_v)
    for i in range(_K // 16):
      one_v[pl.ds(i * 16, 16)] = jnp.ones((16,), jnp.float32)
    plsc.subcore_barrier()

    @pl.loop(0, _RPW)
    def _vals(j):
      for i in range(_K // 16):
        cv = col_v[j, pl.ds(i * 16, 16)]
        val_f[j, pl.ds(i * 16, 16)] = plsc.load_gather(hewt_v, [cv])

    def _drain1(sem):
      pltpu.make_async_copy(zeros1_hbm.at[pl.ds(0, _K)], one_v, sem).wait()

    @pl.loop(0, _RPW // 8)
    def _grp(g):
      for t in range(8):
        j = g * 8 + t
        pltpu.async_copy(val_f.at[j], dd_sh.at[row_v.at[j]], sd, add=True)
        pltpu.async_copy(one_v, bd_sh.at[col_v.at[j]], sb, add=True)
      for t in range(8):
        _drain1(sd)
        _drain1(sb)

    plsc.subcore_barrier()
    pltpu.sync_copy(dd_sh.at[pl.ds(s * d1, d1)], stg_v.at[pl.ds(0, d1)])
    pltpu.sync_copy(stg_v.at[pl.ds(0, d1)], dd_hbm.at[pl.ds(c * t1 + s * d1, d1)])
    pltpu.sync_copy(bd_sh.at[pl.ds(s * d2, d2)], stg_v.at[pl.ds(0, d2)])
    pltpu.sync_copy(stg_v.at[pl.ds(0, d2)], bd_hbm.at[pl.ds(c * t2 + s * d2, d2)])

  return sp


def _make_filterpass():
  t = _ACC_N
  d = t // _NSUB

  @functools.partial(
      pl.kernel,
      out_type=(jax.ShapeDtypeStruct((_NW * _RC, _K), jnp.int32),
                jax.ShapeDtypeStruct((_NW * _RC, _K), jnp.int32),
                jax.ShapeDtypeStruct((_NCORE * t,), jnp.float32),
                jax.ShapeDtypeStruct((_NCORE * t,), jnp.float32),
                jax.ShapeDtypeStruct((_NW * 16,), jnp.int32)),
      mesh=_mesh(),
      compiler_params=pltpu.CompilerParams(needs_layout_passes=False),
      scratch_types=[
          pltpu.VMEM((_RPW, _K), jnp.int32),
          pltpu.VMEM((_RPW, _K), jnp.int32),
          pltpu.VMEM((_RC * _K,), jnp.int32),
          pltpu.VMEM((_RC * _K,), jnp.int32),
          pltpu.VMEM((_RC * _K,), jnp.float32),
          pltpu.VMEM((_RC, _K), jnp.int32),
          pltpu.VMEM((_RC, _K), jnp.int32),
          pltpu.VMEM((_RC, _K), jnp.float32),
          pltpu.VMEM((_SPAD,), jnp.int32),
          pltpu.VMEM((_SPAD,), jnp.float32),
          pltpu.VMEM((_K,), jnp.float32),
          pltpu.VMEM((d,), jnp.float32),
          pltpu.VMEM((16,), jnp.int32),
          pltpu.VMEM_SHARED((t,), jnp.float32),
          pltpu.VMEM_SHARED((t,), jnp.float32),
          pltpu.SemaphoreType.DMA,
          pltpu.SemaphoreType.DMA,
      ],
  )
  def fk(row_hbm, col_hbm, tbl_hbm, hewt_hbm, zeros1_hbm,
         nr_hbm, nc_hbm, dd_hbm, bd_hbm, cnt_hbm,
         row_v, col_v, nr1, nc1, vl1, nr_v, nc_v, vl_v, tbl_v, hewt_v,
         one_v, stg_v, cnt_v, dd_sh, bd_sh, sd, sb):
    c = lax.axis_index("c")
    s = lax.axis_index("s")
    w = s * _NCORE + c

    pltpu.sync_copy(zeros1_hbm.at[pl.ds(0, d)], stg_v)
    pltpu.sync_copy(stg_v, dd_sh.at[pl.ds(s * d, d)])
    pltpu.sync_copy(stg_v, bd_sh.at[pl.ds(s * d, d)])
    pltpu.sync_copy(tbl_hbm, tbl_v)
    pltpu.sync_copy(hewt_hbm, hewt_v)
    pltpu.sync_copy(row_hbm.at[pl.ds(w * _RPW, _RPW)], row_v)
    pltpu.sync_copy(col_hbm.at[pl.ds(w * _RPW, _RPW)], col_v)
    for i in range(_K // 16):
      one_v[pl.ds(i * 16, 16)] = jnp.ones((16,), jnp.float32)
    lanes = lax.iota(jnp.int32, 16)
    plsc.subcore_barrier()

    def _body(j, off):
      for i in range(_K // 16):
        rv = row_v[j, pl.ds(i * 16, 16)]
        cv = col_v[j, pl.ds(i * 16, 16)]
        tr = plsc.load_gather(tbl_v, [rv])
        tc = plsc.load_gather(tbl_v, [cv])
        keep = (tr >= 0) & (tc >= 0)
        plsc.store_compressed(nr1.at[pl.ds(off, 16)], tr, mask=keep)
        plsc.store_compressed(nc1.at[pl.ds(off, 16)], tc, mask=keep)
        wv = plsc.load_gather(hewt_v, [tc], mask=keep)
        plsc.store_compressed(vl1.at[pl.ds(off, 16)], wv, mask=keep)
        off = off + jnp.sum(keep.astype(jnp.int32))
      return off

    off = pl.loop(0, _RPW, init_carry=jnp.int32(0))(_body)

    rem = off & 1023
    nfill = (1024 - rem) & 1023
    nst = (nfill + 15) >> 4

    @pl.loop(0, nst)
    def _fill(i):
      base = off + i * 16
      sent = _N + ((base + lanes) & (_PAD - 1))
      nr1[pl.ds(base, 16)] = sent
      nc1[pl.ds(base, 16)] = sent
      vl1[pl.ds(base, 16)] = jnp.zeros((16,), jnp.float32)

    kgrp = (off + 1023) >> 10
    cnt_v[...] = jnp.full((16,), kgrp, jnp.int32)
    pltpu.sync_copy(cnt_v, cnt_hbm.at[pl.ds(w * 16, 16)])

    @pl.loop(0, kgrp * 8)
    def _re(r):
      for i in range(_K // 16):
        nr_v[r, pl.ds(i * 16, 16)] = nr1[pl.ds(r * _K + i * 16, 16)]
        nc_v[r, pl.ds(i * 16, 16)] = nc1[pl.ds(r * _K + i * 16, 16)]
        vl_v[r, pl.ds(i * 16, 16)] = vl1[pl.ds(r * _K + i * 16, 16)]

    pltpu.sync_copy(nr_v, nr_hbm.at[pl.ds(w * _RC, _RC)])
    pltpu.sync_copy(nc_v, nc_hbm.at[pl.ds(w * _RC, _RC)])

    def _drain1(sem):
      pltpu.make_async_copy(zeros1_hbm.at[pl.ds(0, _K)], one_v, sem).wait()

    @pl.loop(0, kgrp)
    def _grp(g):
      for u in range(8):
        j = g * 8 + u
        pltpu.async_copy(vl_v.at[j], dd_sh.at[nr_v.at[j]], sd, add=True)
        pltpu.async_copy(one_v, bd_sh.at[nc_v.at[j]], sb, add=True)
      for u in range(8):
        _drain1(sd)
        _drain1(sb)

    plsc.subcore_barrier()
    pltpu.sync_copy(dd_sh.at[pl.ds(s * d, d)], stg_v)
    pltpu.sync_copy(stg_v, dd_hbm.at[pl.ds(c * t + s * d, d)])
    pltpu.sync_copy(bd_sh.at[pl.ds(s * d, d)], stg_v)
    pltpu.sync_copy(stg_v, bd_hbm.at[pl.ds(c * t + s * d, d)])

  return fk


_sc_cache = {}


def _feat_m(*args):
  if "fm" not in _sc_cache:
    _sc_cache["fm"] = _make_featpass(_ACC_M, _RPW)
  return _sc_cache["fm"](*args)


def _feat_n(*args):
  if "fn" not in _sc_cache:
    _sc_cache["fn"] = _make_featpass(_ACC_N, _RPW)
  return _sc_cache["fn"](*args)


def _feat_s(*args):
  if "fs" not in _sc_cache:
    _sc_cache["fs"] = _make_featpass(_ACC_N, _RPW_S)
  return _sc_cache["fs"](*args)


def _feat_d(*args):
  if "fd" not in _sc_cache:
    _sc_cache["fd"] = _make_featpass(_ACC_N, _RC, dynamic=True)
  return _sc_cache["fd"](*args)


def _scal_1(*args):
  if "s1" not in _sc_cache:
    _sc_cache["s1"] = _make_scalarpass(_ACC_N, _ACC_M)
  return _sc_cache["s1"](*args)


def _filter2(*args):
  if "f2" not in _sc_cache:
    _sc_cache["f2"] = _make_filterpass()
  return _sc_cache["f2"](*args)


_BR = 1000


def _dense2(a, w1, w2, b2):
  def body(a_ref, w1_ref, w2_ref, b2_ref, o1_ref, o2_ref):
    ab = a_ref[...]
    o1_ref[...] = jnp.dot(ab, w1_ref[...], preferred_element_type=jnp.float32)
    t = jnp.dot(ab, w2_ref[...], preferred_element_type=jnp.float32) + b2_ref[...]
    o2_ref[...] = jnp.where(t > 0, t, _SLOPE * t)

  grid = (_N // _BR,)
  return pl.pallas_call(
      body,
      grid=grid,
      in_specs=[
          pl.BlockSpec((_BR, _H), lambda i: (i, 0)),
          pl.BlockSpec((_H, _H), lambda i: (0, 0)),
          pl.BlockSpec((_H, _H), lambda i: (0, 0)),
          pl.BlockSpec((1, _H), lambda i: (0, 0)),
      ],
      out_specs=[
          pl.BlockSpec((_BR, _H), lambda i: (i, 0)),
          pl.BlockSpec((_BR, _H), lambda i: (i, 0)),
      ],
      out_shape=[
          jax.ShapeDtypeStruct((_N, _H), jnp.float32),
          jax.ShapeDtypeStruct((_N, _H), jnp.float32),
      ],
  )(a, w1, w2, b2.reshape(1, _H))


def _score_tc(agg, z, wrel, wroot, brel):
  wr = jnp.zeros((_H, _H), jnp.float32).at[:, 0].set(wrel.reshape(-1))
  wo = jnp.zeros((_H, _H), jnp.float32).at[:, 0].set(wroot.reshape(-1))

  def body(a_ref, z_ref, wr_ref, wo_ref, b_ref, o_ref):
    t = (jnp.dot(a_ref[...], wr_ref[...], preferred_element_type=jnp.float32)
         + jnp.dot(z_ref[...], wo_ref[...], preferred_element_type=jnp.float32)
         + b_ref[0, 0])
    o_ref[...] = jnp.tanh(t)

  out = pl.pallas_call(
      body,
      grid=(_N // _BR,),
      in_specs=[
          pl.BlockSpec((_BR, _H), lambda i: (i, 0)),
          pl.BlockSpec((_BR, _H), lambda i: (i, 0)),
          pl.BlockSpec((_H, _H), lambda i: (0, 0)),
          pl.BlockSpec((_H, _H), lambda i: (0, 0)),
          pl.BlockSpec((1, 1), lambda i: (0, 0), memory_space=pltpu.SMEM),
      ],
      out_specs=pl.BlockSpec((_BR, _H), lambda i: (i, 0)),
      out_shape=jax.ShapeDtypeStruct((_N, _H), jnp.float32),
  )(agg, z, wr, wo, brel.reshape(1, 1))
  return out[:, 0]


def _head_tc(o, w1, b1, w2, b2):
  w1p = jnp.zeros((_H, _H), jnp.float32).at[:, :w1.shape[1]].set(w1)
  b1p = jnp.zeros((1, _H), jnp.float32).at[0, :b1.shape[0]].set(b1)
  w2p = jnp.zeros((_H, _H), jnp.float32).at[:w2.shape[0], :w2.shape[1]].set(w2)
  b2p = jnp.zeros((1, _H), jnp.float32).at[0, :b2.shape[0]].set(b2)

  def body(o_ref, w1_ref, b1_ref, w2_ref, b2_ref, out_ref):
    t = jnp.dot(o_ref[...], w1_ref[...], preferred_element_type=jnp.float32) + b1_ref[...]
    h = jnp.where(t > 0, t, _SLOPE * t)
    out_ref[...] = jnp.dot(h, w2_ref[...], preferred_element_type=jnp.float32) + b2_ref[...]

  out = pl.pallas_call(
      body,
      out_shape=jax.ShapeDtypeStruct((_G, _H), jnp.float32),
  )(o, w1p, b1p, w2p, b2p)
  return out[:, :w2.shape[1]]


def _lrelu(v):
  return jnp.where(v > 0, v, _SLOPE * v)


def _pad_edges(idx, base):
  pad = base + (jnp.arange(_EPAD - _E, dtype=jnp.int32) & (_PAD - 1))
  return jnp.concatenate([idx, pad]).reshape(_RTOT, _K)


def _pad_rows(a):
  return jnp.pad(a, ((0, _SPAD - a.shape[0]), (0, 0)))


def _onehot_readout(z, b, valid_w, cnt):
  oh = (b[None, :] == jnp.arange(_G, dtype=b.dtype)[:, None]).astype(jnp.float32)
  oh = oh * valid_w[None, :]
  mean = (oh @ z) / jnp.maximum(cnt.astype(jnp.float32), 1.0)[:, None]
  bm = jnp.where(valid_w > 0, b, _G)
  mx = jax.ops.segment_max(z, bm, num_segments=_G)
  mx = jnp.where(jnp.isfinite(mx), mx, 0.0)
  return mean + mx


def _topk_masks(score, seg, counts):
  n = score.shape[0]
  iota = jnp.arange(n)
  order = jnp.lexsort((iota, -score, seg))
  sseg = seg[order]
  starts = jnp.cumsum(counts) - counts
  safe = jnp.minimum(sseg, _G - 1)
  k = jnp.ceil(0.5 * counts.astype(score.dtype)).astype(counts.dtype)
  pos = iota - starts[safe]
  valid = (sseg < _G) & (pos < k[safe])
  return order, valid, k


def _sorted_counts(seg):
  edges = jnp.searchsorted(seg, jnp.arange(_G + 1, dtype=seg.dtype), side="left")
  return (edges[1:] - edges[:-1]).astype(jnp.int32)


def kernel(x, hyperedge_index, hyperedge_weight, batch,
           theta1, hb1, gamma1, beta1, wlin1, blin1, wrel1, brel1, wroot1,
           theta2, hb2, gamma2, beta2, wlin2, blin2, wrel2, brel2, wroot2,
           w_out1, b_out1, w_out2, b_out2):
  row = hyperedge_index[0].astype(jnp.int32)
  col = hyperedge_index[1].astype(jnp.int32)
  zeros2d = jnp.zeros((_CR, _H), jnp.float32)
  zeros1d = jnp.zeros((_SPAD,), jnp.float32)

  row_g = _pad_edges(row, _N)
  col_g = _pad_edges(col, _N)
  col_sc = _pad_edges(col, _M)

  xt, xlin = _dense2(x, theta1, wlin1, blin1)

  he_p = _feat_m(_pad_rows(xt), row_g, col_sc, zeros2d)
  he_sum = he_p[0, :_M] + he_p[1, :_M]
  hewt1 = jnp.zeros((_SPAD,), jnp.float32).at[:_M].set(hyperedge_weight)
  dd_p, bd_p = _scal_1(row_g, col_sc, hewt1, zeros1d)
  dd_p = dd_p.reshape(_NCORE, _ACC_N)
  bd_p = bd_p.reshape(_NCORE, _ACC_M)
  dd = dd_p[0, :_N] + dd_p[1, :_N]
  bd = bd_p[0, :_M] + bd_p[1, :_M]
  dinv = jnp.where(dd > 0, 1.0 / dd, 0.0)
  binv = jnp.where(bd > 0, 1.0 / bd, 0.0)
  he_scaled = he_sum * (binv * hyperedge_weight)[:, None]
  out_p = _feat_n(_pad_rows(he_scaled), col_g, row_g, zeros2d)
  conv1 = (out_p[0, :_N] + out_p[1, :_N]) * dinv[:, None] + hb1

  mu = conv1.mean(0)
  var = conv1.var(0)
  z = _lrelu((conv1 - mu) / jnp.sqrt(var + _EPS) * gamma1 + beta1) + xlin

  agg_p = _feat_m(_pad_rows(z), row_g, col_sc, zeros2d)
  agg = jnp.pad(agg_p[0, :_M] + agg_p[1, :_M], ((0, _N - _M), (0, 0)))
  s1 = _score_tc(agg, z, wrel1, wroot1, brel1)

  counts1 = _sorted_counts(batch)
  order1, valid1, k1g = _topk_masks(s1, batch, counts1)
  compact1 = jnp.cumsum(valid1) - 1
  iota_n = jnp.arange(_N, dtype=jnp.int32)
  spread_n = _N + (iota_n & (_PAD - 1))
  dest1 = jnp.where(valid1, compact1.astype(jnp.int32), spread_n)
  tbl = jnp.full((_SPAD,), -1, jnp.int32).at[order1].set(
      jnp.where(valid1, compact1.astype(jnp.int32), -1))
  k1 = jnp.sum(valid1)
  kcum1 = jnp.cumsum(k1g)
  b1 = jnp.searchsorted(kcum1, iota_n, side="right").astype(batch.dtype)

  pad_s = _N + (jnp.arange(_EPAD_S - _N, dtype=jnp.int32) & (_PAD - 1))
  oidx = jnp.concatenate([order1.astype(jnp.int32), pad_s]).reshape(_RTOT_S, _K)
  didx = jnp.concatenate([dest1, pad_s]).reshape(_RTOT_S, _K)
  z1_p = _feat_s(_pad_rows(z * s1[:, None]), oidx, didx, zeros2d)
  z1 = z1_p[0, :_N] + z1_p[1, :_N]

  rmask = (iota_n < k1).astype(jnp.float32)
  out_r = _onehot_readout(z1, b1, rmask, k1g)

  hew_ext = jnp.concatenate(
      [hyperedge_weight,
       jnp.full((_N - _M,), hyperedge_weight[_M - 1], jnp.float32),
       jnp.zeros((_SPAD - _N,), jnp.float32)])
  nr2, nc2, dd2_p, bd2_p, cnt2 = _filter2(row_g, col_g, tbl, hew_ext, zeros1d)
  dd2_p = dd2_p.reshape(_NCORE, _ACC_N)
  bd2_p = bd2_p.reshape(_NCORE, _ACC_N)
  dd2 = dd2_p[0, :_N] + dd2_p[1, :_N]
  bd2 = bd2_p[0, :_N] + bd2_p[1, :_N]
  dinv2 = jnp.where(dd2 > 0, 1.0 / dd2, 0.0)
  binv2 = jnp.where(bd2 > 0, 1.0 / bd2, 0.0)

  xt2, xlin2 = _dense2(z1, theta2, wlin2, blin2)
  he2_p = _feat_d(_pad_rows(xt2), nr2, nc2, cnt2, zeros2d)
  he2 = (he2_p[0, :_N] + he2_p[1, :_N]) * (binv2 * hew_ext[:_N])[:, None]
  out2_p = _feat_d(_pad_rows(he2), nc2, nr2, cnt2, zeros2d)
  conv2 = (out2_p[0, :_N] + out2_p[1, :_N]) * dinv2[:, None] + hb2

  cmax = jnp.maximum(k1.astype(jnp.float32), 1.0)
  mu2 = (conv2 * rmask[:, None]).sum(0) / cmax
  var2 = (((conv2 - mu2) * rmask[:, None]) ** 2).sum(0) / cmax
  z2l = (_lrelu((conv2 - mu2) / jnp.sqrt(var2 + _EPS) * gamma2 + beta2)
         + xlin2) * rmask[:, None]

  agg2_p = _feat_d(_pad_rows(z2l), nr2, nc2, cnt2, zeros2d)
  agg2 = agg2_p[0, :_N] + agg2_p[1, :_N]
  s2 = _score_tc(agg2, z2l, wrel2, wroot2, brel2)

  counts2 = _sorted_counts(b1)
  order2, valid2, k2g = _topk_masks(s2, b1, counts2)
  vslot = jnp.zeros((_N,), jnp.bool_).at[order2].set(valid2)
  w2v = jnp.where(vslot, s2, 0.0)
  zw2 = z2l * w2v[:, None]
  out_r = out_r + _onehot_readout(zw2, b1, vslot.astype(jnp.float32), k2g)

  return _head_tc(out_r, w_out1, b_out1, w_out2, b_out2)


# speedup vs baseline: 23.0049x; 1.1277x over previous
import functools

import jax
import jax.numpy as jnp
from jax import lax
from jax.experimental import pallas as pl
from jax.experimental.pallas import tpu as pltpu
from jax.experimental.pallas import tpu_sc as plsc

_N = 10000
_M = 6000
_E = 320000
_H = 128
_G = 16
_SLOPE = 0.01
_EPS = 1e-5

_NCORE = 2
_NSUB = 16
_NW = _NCORE * _NSUB

_K = 128
_RPW = 80
_RTOT = _RPW * _NW
_EPAD = _RTOT * _K

_RPW_S = 8
_RTOT_S = _RPW_S * _NW
_EPAD_S = _RTOT_S * _K

_RC = 88

_PAD = 1024
_SPAD = 11264
_ACC_M = 7168
_ACC_N = 11264
_CR = 64


def _mesh():
  return plsc.VectorSubcoreMesh(
      core_axis_name="c", subcore_axis_name="s",
      num_cores=_NCORE, num_subcores=_NSUB)


def _make_featpass(tacc, rpw, dynamic=False):
  rows_per_tile = tacc // _NSUB
  nchunk = rows_per_tile // _CR
  in_cnt = ([jax.ShapeDtypeStruct((_NW * 16,), jnp.int32)] if dynamic else [])

  def make(body):
    return pl.kernel(
        body,
        out_type=jax.ShapeDtypeStruct((_NCORE, tacc, _H), jnp.float32),
        mesh=_mesh(),
        scratch_types=[
            pltpu.VMEM((8, _K), jnp.int32),
            pltpu.VMEM((8, _K), jnp.int32),
            pltpu.VMEM((16,), jnp.int32),
            pltpu.VMEM((_K, _H), jnp.float32),
            pltpu.VMEM((_K, _H), jnp.float32),
            pltpu.VMEM_SHARED((tacc, _H), jnp.float32),
            pltpu.SemaphoreType.DMA,
            pltpu.SemaphoreType.DMA,
            pltpu.SemaphoreType.DMA,
            pltpu.SemaphoreType.DMA,
        ])

  def fp_body(src_hbm, idxin_hbm, idxout_hbm, *rest):
    if dynamic:
      cnt_hbm = rest[0]
      rest = rest[1:]
    (zeros_hbm, out_hbm,
     idxi_v, idxo_v, cnt_v, rows_v, rows2_v, acc_sh, ga, gb, sa, sb) = rest
    c = lax.axis_index("c")
    s = lax.axis_index("s")
    w = s * _NCORE + c
    base0 = s * rows_per_tile

    pltpu.sync_copy(zeros_hbm, rows_v.at[pl.ds(0, _CR)])

    @pl.loop(0, nchunk)
    def _zero(i):
      pltpu.sync_copy(rows_v.at[pl.ds(0, _CR)], acc_sh.at[pl.ds(base0 + i * _CR, _CR)])

    plsc.subcore_barrier()

    def _drain(buf, sem):
      pltpu.make_async_copy(src_hbm.at[pl.ds(0, _K)], buf, sem).wait()

    if dynamic:
      pltpu.sync_copy(cnt_hbm.at[pl.ds(w * 16, 16)], cnt_v)
      ngrp = cnt_v[...][0]
    else:
      ngrp = rpw // 8

    @pl.loop(0, ngrp)
    def _grp(g):
      gbase = w * rpw + g * 8
      pltpu.sync_copy(idxin_hbm.at[pl.ds(gbase, 8)], idxi_v)
      pltpu.sync_copy(idxout_hbm.at[pl.ds(gbase, 8)], idxo_v)
      pltpu.async_copy(src_hbm.at[idxi_v.at[0]], rows_v, ga)
      for jj in range(4):
        j = jj * 2
        _drain(rows_v, ga)
        pltpu.async_copy(src_hbm.at[idxi_v.at[j + 1]], rows2_v, gb)
        pltpu.async_copy(rows_v, acc_sh.at[idxo_v.at[j]], sa, add=True)
        _drain(rows2_v, gb)
        _drain(rows_v, sa)
        if j + 2 < 8:
          pltpu.async_copy(src_hbm.at[idxi_v.at[j + 2]], rows_v, ga)
        pltpu.async_copy(rows2_v, acc_sh.at[idxo_v.at[j + 1]], sb, add=True)
        _drain(rows2_v, sb)

    plsc.subcore_barrier()

    @pl.loop(0, nchunk)
    def _out(i):
      off = base0 + i * _CR
      pltpu.sync_copy(acc_sh.at[pl.ds(off, _CR)], rows_v.at[pl.ds(0, _CR)])
      pltpu.sync_copy(rows_v.at[pl.ds(0, _CR)], out_hbm.at[c, pl.ds(off, _CR)])

  return make(fp_body)


def _make_scalarpass(t1, t2):
  d1 = t1 // _NSUB
  d2 = t2 // _NSUB

  @functools.partial(
      pl.kernel,
      out_type=(jax.ShapeDtypeStruct((_NCORE * t1,), jnp.float32),
                jax.ShapeDtypeStruct((_NCORE * t2,), jnp.float32)),
      mesh=_mesh(),
      compiler_params=pltpu.CompilerParams(needs_layout_passes=False),
      scratch_types=[
          pltpu.VMEM((_RPW, _K), jnp.int32),
          pltpu.VMEM((_RPW, _K), jnp.int32),
          pltpu.VMEM((_SPAD,), jnp.float32),
          pltpu.VMEM((_RPW, _K), jnp.float32),
          pltpu.VMEM((_K,), jnp.float32),
          pltpu.VMEM((max(t1, t2) // _NSUB,), jnp.float32),
          pltpu.VMEM_SHARED((t1,), jnp.float32),
          pltpu.VMEM_SHARED((t2,), jnp.float32),
          pltpu.SemaphoreType.DMA,
          pltpu.SemaphoreType.DMA,
      ],
  )
  def sp(row_hbm, col_hbm, hewt_hbm, zeros1_hbm, dd_hbm, bd_hbm,
         row_v, col_v, hewt_v, val_f, one_v, stg_v, dd_sh, bd_sh, sd, sb):
    c = lax.axis_index("c")
    s = lax.axis_index("s")
    w = s * _NCORE + c

    pltpu.sync_copy(zeros1_hbm.at[pl.ds(0, d1)], stg_v.at[pl.ds(0, d1)])
    pltpu.sync_copy(stg_v.at[pl.ds(0, d1)], dd_sh.at[pl.ds(s * d1, d1)])
    pltpu.sync_copy(stg_v.at[pl.ds(0, d2)], bd_sh.at[pl.ds(s * d2, d2)])
    pltpu.sync_copy(hewt_hbm, hewt_v)
    pltpu.sync_copy(row_hbm.at[pl.ds(w * _RPW, _RPW)], row_v)
    pltpu.sync_copy(col_hbm.at[pl.ds(w * _RPW, _RPW)], col_v)
    for i in range(_K // 16):
      one_v[pl.ds(i * 16, 16)] = jnp.ones((16,), jnp.float32)
    plsc.subcore_barrier()

    @pl.loop(0, _RPW)
    def _vals(j):
      for i in range(_K // 16):
        cv = col_v[j, pl.ds(i * 16, 16)]
        val_f[j, pl.ds(i * 16, 16)] = plsc.load_gather(hewt_v, [cv])

    def _drain1(sem):
      pltpu.make_async_copy(zeros1_hbm.at[pl.ds(0, _K)], one_v, sem).wait()

    @pl.loop(0, _RPW // 8)
    def _grp(g):
      for t in range(8):
        j = g * 8 + t
        pltpu.async_copy(val_f.at[j], dd_sh.at[row_v.at[j]], sd, add=True)
        pltpu.async_copy(one_v, bd_sh.at[col_v.at[j]], sb, add=True)
      for t in range(8):
        _drain1(sd)
        _drain1(sb)

    plsc.subcore_barrier()
    pltpu.sync_copy(dd_sh.at[pl.ds(s * d1, d1)], stg_v.at[pl.ds(0, d1)])
    pltpu.sync_copy(stg_v.at[pl.ds(0, d1)], dd_hbm.at[pl.ds(c * t1 + s * d1, d1)])
    pltpu.sync_copy(bd_sh.at[pl.ds(s * d2, d2)], stg_v.at[pl.ds(0, d2)])
    pltpu.sync_copy(stg_v.at[pl.ds(0, d2)], bd_hbm.at[pl.ds(c * t2 + s * d2, d2)])

  return sp


def _make_tabscatter(t, dynamic=False):
  d = t // _NSUB
  rpw = _RC if dynamic else _RPW
  in_cnt = ([jax.ShapeDtypeStruct((_NW * 16,), jnp.int32)] if dynamic else [])

  @functools.partial(
      pl.kernel,
      out_type=jax.ShapeDtypeStruct((_NCORE * t,), jnp.float32),
      mesh=_mesh(),
      compiler_params=pltpu.CompilerParams(needs_layout_passes=False),
      scratch_types=[
          pltpu.VMEM((rpw, _K), jnp.int32),
          pltpu.VMEM((rpw, _K), jnp.int32),
          pltpu.VMEM((_SPAD,), jnp.float32),
          pltpu.VMEM((rpw, _K), jnp.float32),
          pltpu.VMEM((_K,), jnp.float32),
          pltpu.VMEM((d,), jnp.float32),
          pltpu.VMEM((16,), jnp.int32),
          pltpu.VMEM_SHARED((t,), jnp.float32),
          pltpu.SemaphoreType.DMA,
      ],
  )
  def ts(idxg_hbm, idxs_hbm, tab_hbm, zeros1_hbm, *rest):
    if dynamic:
      cnt_hbm = rest[0]
      rest = rest[1:]
    (acc_hbm, idxg_v, idxs_v, tab_v, val_f, drn_v, stg_v, cnt_v, acc_sh,
     sd) = rest
    c = lax.axis_index("c")
    s = lax.axis_index("s")
    w = s * _NCORE + c

    pltpu.sync_copy(zeros1_hbm.at[pl.ds(0, d)], stg_v)
    pltpu.sync_copy(stg_v, acc_sh.at[pl.ds(s * d, d)])
    pltpu.sync_copy(tab_hbm, tab_v)
    pltpu.sync_copy(idxg_hbm.at[pl.ds(w * rpw, rpw)], idxg_v)
    pltpu.sync_copy(idxs_hbm.at[pl.ds(w * rpw, rpw)], idxs_v)
    if dynamic:
      pltpu.sync_copy(cnt_hbm.at[pl.ds(w * 16, 16)], cnt_v)
      ngrp = cnt_v[...][0]
    else:
      ngrp = rpw // 8
    plsc.subcore_barrier()

    @pl.loop(0, ngrp * 8)
    def _vals(j):
      for i in range(_K // 16):
        gv = idxg_v[j, pl.ds(i * 16, 16)]
        val_f[j, pl.ds(i * 16, 16)] = plsc.load_gather(tab_v, [gv])

    def _drain1(sem):
      pltpu.make_async_copy(zeros1_hbm.at[pl.ds(0, _K)], drn_v, sem).wait()

    @pl.loop(0, ngrp)
    def _grp(g):
      for u in range(8):
        j = g * 8 + u
        pltpu.async_copy(val_f.at[j], acc_sh.at[idxs_v.at[j]], sd, add=True)
      for u in range(8):
        _drain1(sd)

    plsc.subcore_barrier()
    pltpu.sync_copy(acc_sh.at[pl.ds(s * d, d)], stg_v)
    pltpu.sync_copy(stg_v, acc_hbm.at[pl.ds(c * t + s * d, d)])

  return ts


def _make_filterpass():
  t = _ACC_N
  d = t // _NSUB

  @functools.partial(
      pl.kernel,
      out_type=(jax.ShapeDtypeStruct((_NW * _RC, _K), jnp.int32),
                jax.ShapeDtypeStruct((_NW * _RC, _K), jnp.int32),
                jax.ShapeDtypeStruct((_NCORE * t,), jnp.float32),
                jax.ShapeDtypeStruct((_NCORE * t,), jnp.float32),
                jax.ShapeDtypeStruct((_NW * 16,), jnp.int32)),
      mesh=_mesh(),
      compiler_params=pltpu.CompilerParams(needs_layout_passes=False),
      scratch_types=[
          pltpu.VMEM((_RPW, _K), jnp.int32),
          pltpu.VMEM((_RPW, _K), jnp.int32),
          pltpu.VMEM((_RC * _K,), jnp.int32),
          pltpu.VMEM((_RC * _K,), jnp.int32),
          pltpu.VMEM((_RC * _K,), jnp.float32),
          pltpu.VMEM((_RC, _K), jnp.int32),
          pltpu.VMEM((_RC, _K), jnp.int32),
          pltpu.VMEM((_RC, _K), jnp.float32),
          pltpu.VMEM((_SPAD,), jnp.int32),
          pltpu.VMEM((_SPAD,), jnp.float32),
          pltpu.VMEM((_K,), jnp.float32),
          pltpu.VMEM((d,), jnp.float32),
          pltpu.VMEM((16,), jnp.int32),
          pltpu.VMEM_SHARED((t,), jnp.float32),
          pltpu.VMEM_SHARED((t,), jnp.float32),
          pltpu.SemaphoreType.DMA,
          pltpu.SemaphoreType.DMA,
      ],
  )
  def fk(row_hbm, col_hbm, tbl_hbm, hewt_hbm, zeros1_hbm,
         nr_hbm, nc_hbm, dd_hbm, bd_hbm, cnt_hbm,
         row_v, col_v, nr1, nc1, vl1, nr_v, nc_v, vl_v, tbl_v, hewt_v,
         one_v, stg_v, cnt_v, dd_sh, bd_sh, sd, sb):
    c = lax.axis_index("c")
    s = lax.axis_index("s")
    w = s * _NCORE + c

    pltpu.sync_copy(zeros1_hbm.at[pl.ds(0, d)], stg_v)
    pltpu.sync_copy(stg_v, dd_sh.at[pl.ds(s * d, d)])
    pltpu.sync_copy(stg_v, bd_sh.at[pl.ds(s * d, d)])
    pltpu.sync_copy(tbl_hbm, tbl_v)
    pltpu.sync_copy(hewt_hbm, hewt_v)
    pltpu.sync_copy(row_hbm.at[pl.ds(w * _RPW, _RPW)], row_v)
    pltpu.sync_copy(col_hbm.at[pl.ds(w * _RPW, _RPW)], col_v)
    for i in range(_K // 16):
      one_v[pl.ds(i * 16, 16)] = jnp.ones((16,), jnp.float32)
    lanes = lax.iota(jnp.int32, 16)
    plsc.subcore_barrier()

    def _body(j, off):
      for i in range(_K // 16):
        rv = row_v[j, pl.ds(i * 16, 16)]
        cv = col_v[j, pl.ds(i * 16, 16)]
        tr = plsc.load_gather(tbl_v, [rv])
        tc = plsc.load_gather(tbl_v, [cv])
        keep = (tr >= 0) & (tc >= 0)
        plsc.store_compressed(nr1.at[pl.ds(off, 16)], tr, mask=keep)
        plsc.store_compressed(nc1.at[pl.ds(off, 16)], tc, mask=keep)
        wv = plsc.load_gather(hewt_v, [tc], mask=keep)
        plsc.store_compressed(vl1.at[pl.ds(off, 16)], wv, mask=keep)
        off = off + jnp.sum(keep.astype(jnp.int32))
      return off

    off = pl.loop(0, _RPW, init_carry=jnp.int32(0))(_body)

    rem = off & 1023
    nfill = (1024 - rem) & 1023
    nst = (nfill + 15) >> 4

    @pl.loop(0, nst)
    def _fill(i):
      base = off + i * 16
      sent = _N + ((base + lanes) & (_PAD - 1))
      nr1[pl.ds(base, 16)] = sent
      nc1[pl.ds(base, 16)] = sent
      vl1[pl.ds(base, 16)] = jnp.zeros((16,), jnp.float32)

    kgrp = (off + 1023) >> 10
    cnt_v[...] = jnp.full((16,), kgrp, jnp.int32)
    pltpu.sync_copy(cnt_v, cnt_hbm.at[pl.ds(w * 16, 16)])

    @pl.loop(0, kgrp * 8)
    def _re(r):
      for i in range(_K // 16):
        nr_v[r, pl.ds(i * 16, 16)] = nr1[pl.ds(r * _K + i * 16, 16)]
        nc_v[r, pl.ds(i * 16, 16)] = nc1[pl.ds(r * _K + i * 16, 16)]
        vl_v[r, pl.ds(i * 16, 16)] = vl1[pl.ds(r * _K + i * 16, 16)]

    pltpu.sync_copy(nr_v, nr_hbm.at[pl.ds(w * _RC, _RC)])
    pltpu.sync_copy(nc_v, nc_hbm.at[pl.ds(w * _RC, _RC)])

    def _drain1(sem):
      pltpu.make_async_copy(zeros1_hbm.at[pl.ds(0, _K)], one_v, sem).wait()

    @pl.loop(0, kgrp)
    def _grp(g):
      for u in range(8):
        j = g * 8 + u
        pltpu.async_copy(vl_v.at[j], dd_sh.at[nr_v.at[j]], sd, add=True)
        pltpu.async_copy(one_v, bd_sh.at[nc_v.at[j]], sb, add=True)
      for u in range(8):
        _drain1(sd)
        _drain1(sb)

    plsc.subcore_barrier()
    pltpu.sync_copy(dd_sh.at[pl.ds(s * d, d)], stg_v)
    pltpu.sync_copy(stg_v, dd_hbm.at[pl.ds(c * t + s * d, d)])
    pltpu.sync_copy(bd_sh.at[pl.ds(s * d, d)], stg_v)
    pltpu.sync_copy(stg_v, bd_hbm.at[pl.ds(c * t + s * d, d)])

  return fk


_sc_cache = {}


def _feat_m(*args):
  if "fm" not in _sc_cache:
    _sc_cache["fm"] = _make_featpass(_ACC_M, _RPW)
  return _sc_cache["fm"](*args)


def _feat_n(*args):
  if "fn" not in _sc_cache:
    _sc_cache["fn"] = _make_featpass(_ACC_N, _RPW)
  return _sc_cache["fn"](*args)


def _feat_s(*args):
  if "fs" not in _sc_cache:
    _sc_cache["fs"] = _make_featpass(_ACC_N, _RPW_S)
  return _sc_cache["fs"](*args)


def _feat_d(*args):
  if "fd" not in _sc_cache:
    _sc_cache["fd"] = _make_featpass(_ACC_N, _RC, dynamic=True)
  return _sc_cache["fd"](*args)


def _scal_1(*args):
  if "s1" not in _sc_cache:
    _sc_cache["s1"] = _make_scalarpass(_ACC_N, _ACC_M)
  return _sc_cache["s1"](*args)


def _filter2(*args):
  if "f2" not in _sc_cache:
    _sc_cache["f2"] = _make_filterpass()
  return _sc_cache["f2"](*args)


def _tabs_m(*args):
  if "tm" not in _sc_cache:
    _sc_cache["tm"] = _make_tabscatter(_ACC_M)
  return _sc_cache["tm"](*args)


def _tabs_d(*args):
  if "td" not in _sc_cache:
    _sc_cache["td"] = _make_tabscatter(_ACC_N, dynamic=True)
  return _sc_cache["td"](*args)


_BR = 1000


def _dense2(a, w1, w2, b2):
  def body(a_ref, w1_ref, w2_ref, b2_ref, o1_ref, o2_ref):
    ab = a_ref[...]
    o1_ref[...] = jnp.dot(ab, w1_ref[...], preferred_element_type=jnp.float32)
    t = jnp.dot(ab, w2_ref[...], preferred_element_type=jnp.float32) + b2_ref[...]
    o2_ref[...] = jnp.where(t > 0, t, _SLOPE * t)

  grid = (_N // _BR,)
  return pl.pallas_call(
      body,
      grid=grid,
      in_specs=[
          pl.BlockSpec((_BR, _H), lambda i: (i, 0)),
          pl.BlockSpec((_H, _H), lambda i: (0, 0)),
          pl.BlockSpec((_H, _H), lambda i: (0, 0)),
          pl.BlockSpec((1, _H), lambda i: (0, 0)),
      ],
      out_specs=[
          pl.BlockSpec((_BR, _H), lambda i: (i, 0)),
          pl.BlockSpec((_BR, _H), lambda i: (i, 0)),
      ],
      out_shape=[
          jax.ShapeDtypeStruct((_N, _H), jnp.float32),
          jax.ShapeDtypeStruct((_N, _H), jnp.float32),
      ],
  )(a, w1, w2, b2.reshape(1, _H))


def _score_tc(agg, z, wrel, wroot, brel):
  wr = jnp.zeros((_H, _H), jnp.float32).at[:, 0].set(wrel.reshape(-1))
  wo = jnp.zeros((_H, _H), jnp.float32).at[:, 0].set(wroot.reshape(-1))

  def body(a_ref, z_ref, wr_ref, wo_ref, b_ref, o_ref):
    t = (jnp.dot(a_ref[...], wr_ref[...], preferred_element_type=jnp.float32)
         + jnp.dot(z_ref[...], wo_ref[...], preferred_element_type=jnp.float32)
         + b_ref[0, 0])
    o_ref[...] = jnp.tanh(t)

  out = pl.pallas_call(
      body,
      grid=(_N // _BR,),
      in_specs=[
          pl.BlockSpec((_BR, _H), lambda i: (i, 0)),
          pl.BlockSpec((_BR, _H), lambda i: (i, 0)),
          pl.BlockSpec((_H, _H), lambda i: (0, 0)),
          pl.BlockSpec((_H, _H), lambda i: (0, 0)),
          pl.BlockSpec((1, 1), lambda i: (0, 0), memory_space=pltpu.SMEM),
      ],
      out_specs=pl.BlockSpec((_BR, _H), lambda i: (i, 0)),
      out_shape=jax.ShapeDtypeStruct((_N, _H), jnp.float32),
  )(agg, z, wr, wo, brel.reshape(1, 1))
  return out[:, 0]


def _mv2_tc(z, wrel, wroot):
  wcat = (jnp.zeros((_H, _H), jnp.float32)
          .at[:, 0].set(wrel.reshape(-1)).at[:, 1].set(wroot.reshape(-1)))

  def body(z_ref, w_ref, o_ref):
    o_ref[...] = jnp.dot(z_ref[...], w_ref[...], preferred_element_type=jnp.float32)

  return pl.pallas_call(
      body,
      grid=(_N // _BR,),
      in_specs=[
          pl.BlockSpec((_BR, _H), lambda i: (i, 0)),
          pl.BlockSpec((_H, _H), lambda i: (0, 0)),
      ],
      out_specs=pl.BlockSpec((_BR, _H), lambda i: (i, 0)),
      out_shape=jax.ShapeDtypeStruct((_N, _H), jnp.float32),
  )(z, wcat)


def _head_tc(o, w1, b1, w2, b2):
  w1p = jnp.zeros((_H, _H), jnp.float32).at[:, :w1.shape[1]].set(w1)
  b1p = jnp.zeros((1, _H), jnp.float32).at[0, :b1.shape[0]].set(b1)
  w2p = jnp.zeros((_H, _H), jnp.float32).at[:w2.shape[0], :w2.shape[1]].set(w2)
  b2p = jnp.zeros((1, _H), jnp.float32).at[0, :b2.shape[0]].set(b2)

  def body(o_ref, w1_ref, b1_ref, w2_ref, b2_ref, out_ref):
    t = jnp.dot(o_ref[...], w1_ref[...], preferred_element_type=jnp.float32) + b1_ref[...]
    h = jnp.where(t > 0, t, _SLOPE * t)
    out_ref[...] = jnp.dot(h, w2_ref[...], preferred_element_type=jnp.float32) + b2_ref[...]

  out = pl.pallas_call(
      body,
      out_shape=jax.ShapeDtypeStruct((_G, _H), jnp.float32),
  )(o, w1p, b1p, w2p, b2p)
  return out[:, :w2.shape[1]]


def _lrelu(v):
  return jnp.where(v > 0, v, _SLOPE * v)


def _pad_edges(idx, base):
  pad = base + (jnp.arange(_EPAD - _E, dtype=jnp.int32) & (_PAD - 1))
  return jnp.concatenate([idx, pad]).reshape(_RTOT, _K)


def _pad_rows(a):
  return jnp.pad(a, ((0, _SPAD - a.shape[0]), (0, 0)))


def _onehot_readout(z, b, valid_w, cnt):
  oh = (b[None, :] == jnp.arange(_G, dtype=b.dtype)[:, None]).astype(jnp.float32)
  oh = oh * valid_w[None, :]
  mean = (oh @ z) / jnp.maximum(cnt.astype(jnp.float32), 1.0)[:, None]
  bm = jnp.where(valid_w > 0, b, _G)
  mx = jax.ops.segment_max(z, bm, num_segments=_G)
  mx = jnp.where(jnp.isfinite(mx), mx, 0.0)
  return mean + mx


def _topk_masks(score, seg, counts):
  n = score.shape[0]
  iota = jnp.arange(n)
  order = jnp.lexsort((iota, -score, seg))
  sseg = seg[order]
  starts = jnp.cumsum(counts) - counts
  safe = jnp.minimum(sseg, _G - 1)
  k = jnp.ceil(0.5 * counts.astype(score.dtype)).astype(counts.dtype)
  pos = iota - starts[safe]
  valid = (sseg < _G) & (pos < k[safe])
  return order, valid, k


def _sorted_counts(seg):
  edges = jnp.searchsorted(seg, jnp.arange(_G + 1, dtype=seg.dtype), side="left")
  return (edges[1:] - edges[:-1]).astype(jnp.int32)


def kernel(x, hyperedge_index, hyperedge_weight, batch,
           theta1, hb1, gamma1, beta1, wlin1, blin1, wrel1, brel1, wroot1,
           theta2, hb2, gamma2, beta2, wlin2, blin2, wrel2, brel2, wroot2,
           w_out1, b_out1, w_out2, b_out2):
  row = hyperedge_index[0].astype(jnp.int32)
  col = hyperedge_index[1].astype(jnp.int32)
  zeros2d = jnp.zeros((_CR, _H), jnp.float32)
  zeros1d = jnp.zeros((_SPAD,), jnp.float32)

  row_g = _pad_edges(row, _N)
  col_g = _pad_edges(col, _N)
  col_sc = _pad_edges(col, _M)

  xt, xlin = _dense2(x, theta1, wlin1, blin1)

  he_p = _feat_m(_pad_rows(xt), row_g, col_sc, zeros2d)
  he_sum = he_p[0, :_M] + he_p[1, :_M]
  hewt1 = jnp.zeros((_SPAD,), jnp.float32).at[:_M].set(hyperedge_weight)
  dd_p, bd_p = _scal_1(row_g, col_sc, hewt1, zeros1d)
  dd_p = dd_p.reshape(_NCORE, _ACC_N)
  bd_p = bd_p.reshape(_NCORE, _ACC_M)
  dd = dd_p[0, :_N] + dd_p[1, :_N]
  bd = bd_p[0, :_M] + bd_p[1, :_M]
  dinv = jnp.where(dd > 0, 1.0 / dd, 0.0)
  binv = jnp.where(bd > 0, 1.0 / bd, 0.0)
  he_scaled = he_sum * (binv * hyperedge_weight)[:, None]
  out_p = _feat_n(_pad_rows(he_scaled), col_g, row_g, zeros2d)
  conv1 = (out_p[0, :_N] + out_p[1, :_N]) * dinv[:, None] + hb1

  mu = conv1.mean(0)
  var = conv1.var(0)
  z = _lrelu((conv1 - mu) / jnp.sqrt(var + _EPS) * gamma1 + beta1) + xlin

  mv1 = _mv2_tc(z, wrel1, wroot1)
  zw1t = jnp.pad(mv1[:, 0], (0, _SPAD - _N))
  aggc_p = _tabs_m(row_g, col_sc, zw1t, zeros1d).reshape(_NCORE, _ACC_M)
  aggc = jnp.pad(aggc_p[0, :_M] + aggc_p[1, :_M], (0, _N - _M))
  s1 = jnp.tanh(aggc + brel1[0] + mv1[:, 1])

  counts1 = _sorted_counts(batch)
  order1, valid1, k1g = _topk_masks(s1, batch, counts1)
  compact1 = jnp.cumsum(valid1) - 1
  iota_n = jnp.arange(_N, dtype=jnp.int32)
  spread_n = _N + (iota_n & (_PAD - 1))
  dest1 = jnp.where(valid1, compact1.astype(jnp.int32), spread_n)
  tbl = jnp.full((_SPAD,), -1, jnp.int32).at[order1].set(
      jnp.where(valid1, compact1.astype(jnp.int32), -1))
  k1 = jnp.sum(valid1)
  kcum1 = jnp.cumsum(k1g)
  b1 = jnp.searchsorted(kcum1, iota_n, side="right").astype(batch.dtype)

  pad_s = _N + (jnp.arange(_EPAD_S - _N, dtype=jnp.int32) & (_PAD - 1))
  oidx = jnp.concatenate([order1.astype(jnp.int32), pad_s]).reshape(_RTOT_S, _K)
  didx = jnp.concatenate([dest1, pad_s]).reshape(_RTOT_S, _K)
  z1_p = _feat_s(_pad_rows(z * s1[:, None]), oidx, didx, zeros2d)
  z1 = z1_p[0, :_N] + z1_p[1, :_N]

  rmask = (iota_n < k1).astype(jnp.float32)
  out_r = _onehot_readout(z1, b1, rmask, k1g)

  hew_ext = jnp.concatenate(
      [hyperedge_weight,
       jnp.full((_N - _M,), hyperedge_weight[_M - 1], jnp.float32),
       jnp.zeros((_SPAD - _N,), jnp.float32)])
  nr2, nc2, dd2_p, bd2_p, cnt2 = _filter2(row_g, col_g, tbl, hew_ext, zeros1d)
  dd2_p = dd2_p.reshape(_NCORE, _ACC_N)
  bd2_p = bd2_p.reshape(_NCORE, _ACC_N)
  dd2 = dd2_p[0, :_N] + dd2_p[1, :_N]
  bd2 = bd2_p[0, :_N] + bd2_p[1, :_N]
  dinv2 = jnp.where(dd2 > 0, 1.0 / dd2, 0.0)
  binv2 = jnp.where(bd2 > 0, 1.0 / bd2, 0.0)

  xt2, xlin2 = _dense2(z1, theta2, wlin2, blin2)
  he2_p = _feat_d(_pad_rows(xt2), nr2, nc2, cnt2, zeros2d)
  he2 = (he2_p[0, :_N] + he2_p[1, :_N]) * (binv2 * hew_ext[:_N])[:, None]
  out2_p = _feat_d(_pad_rows(he2), nc2, nr2, cnt2, zeros2d)
  conv2 = (out2_p[0, :_N] + out2_p[1, :_N]) * dinv2[:, None] + hb2

  cmax = jnp.maximum(k1.astype(jnp.float32), 1.0)
  mu2 = (conv2 * rmask[:, None]).sum(0) / cmax
  var2 = (((conv2 - mu2) * rmask[:, None]) ** 2).sum(0) / cmax
  z2l = (_lrelu((conv2 - mu2) / jnp.sqrt(var2 + _EPS) * gamma2 + beta2)
         + xlin2) * rmask[:, None]

  mv2 = _mv2_tc(z2l, wrel2, wroot2)
  zw2t = jnp.pad(mv2[:, 0], (0, _SPAD - _N))
  aggc2_p = _tabs_d(nr2, nc2, zw2t, zeros1d, cnt2).reshape(_NCORE, _ACC_N)
  s2 = jnp.tanh(aggc2_p[0, :_N] + aggc2_p[1, :_N] + brel2[0] + mv2[:, 1])

  counts2 = _sorted_counts(b1)
  order2, valid2, k2g = _topk_masks(s2, b1, counts2)
  vslot = jnp.zeros((_N,), jnp.bool_).at[order2].set(valid2)
  w2v = jnp.where(vslot, s2, 0.0)
  zw2 = z2l * w2v[:, None]
  out_r = out_r + _onehot_readout(zw2, b1, vslot.astype(jnp.float32), k2g)

  return _head_tc(out_r, w_out1, b_out1, w_out2, b_out2)
